# baseline (speedup 1.0000x reference)
import functools

import jax
import jax.numpy as jnp
from jax import lax
from jax.experimental import pallas as pl
from jax.experimental.pallas import tpu as pltpu
from jax.experimental.pallas import tpu_sc as plsc

N = 10000
NP = 10240
D = 128
E = 320000
CH = 128
NC = 2
NS = 16
NW = NC * NS
KCH = (-((E + NW * CH - 1) // (NW * CH)) // 8) * -8
EP = NW * KCH * CH
RPS = NP // NS

_mesh = plsc.VectorSubcoreMesh(core_axis_name="c", subcore_axis_name="s")


@functools.partial(
    pl.kernel,
    out_type=(
        jax.ShapeDtypeStruct((NC, NP), jnp.float32),
        jax.ShapeDtypeStruct((NC, NP), jnp.float32),
    ),
    mesh=_mesh,
    scratch_types=[
        pltpu.VMEM((KCH, CH), jnp.int32),
        pltpu.VMEM((KCH, CH), jnp.int32),
        pltpu.VMEM((CH,), jnp.float32),
        pltpu.VMEM((1, CH), jnp.int32),
        pltpu.VMEM_SHARED((NP,), jnp.float32),
        pltpu.VMEM_SHARED((NP,), jnp.float32),
    ],
)
def _sc_hist(row_hbm, col_hbm, zeros1_hbm, degp, selfp,
             rows_v, cols_v, ones_v, sel_v, deg_sh, self_sh):
    c = lax.axis_index("c")
    s = lax.axis_index("s")
    w = s * NC + c
    pltpu.sync_copy(zeros1_hbm.at[pl.ds(s * RPS, RPS)],
                    deg_sh.at[pl.ds(s * RPS, RPS)])
    pltpu.sync_copy(zeros1_hbm.at[pl.ds(s * RPS, RPS)],
                    self_sh.at[pl.ds(s * RPS, RPS)])
    for j in range(CH // 16):
        ones_v[pl.ds(j * 16, 16)] = jnp.ones((16,), jnp.float32)
    pltpu.sync_copy(row_hbm.at[pl.ds(w * KCH, KCH)], rows_v)
    pltpu.sync_copy(col_hbm.at[pl.ds(w * KCH, KCH)], cols_v)
    plsc.subcore_barrier()

    @pl.loop(0, KCH)
    def _(k):
        pltpu.sync_copy(ones_v, deg_sh.at[cols_v.at[k]], add=True)
        for j in range(CH // 16):
            r = rows_v[k, pl.ds(j * 16, 16)]
            cc = cols_v[k, pl.ds(j * 16, 16)]
            sel_v[0, pl.ds(j * 16, 16)] = jnp.where(r == cc, cc, NP - 1)
        pltpu.sync_copy(ones_v, self_sh.at[sel_v.at[0]], add=True)

    plsc.subcore_barrier()
    pltpu.sync_copy(deg_sh.at[pl.ds(s * RPS, RPS)],
                    degp.at[c, pl.ds(s * RPS, RPS)])
    pltpu.sync_copy(self_sh.at[pl.ds(s * RPS, RPS)],
                    selfp.at[c, pl.ds(s * RPS, RPS)])


HALF = KCH // 2


@functools.partial(
    pl.kernel,
    out_type=jax.ShapeDtypeStruct((NC, NP, D), jnp.float32),
    mesh=_mesh,
    scratch_types=[
        pltpu.VMEM((HALF, CH), jnp.int32),
        pltpu.VMEM((HALF, CH), jnp.int32),
        pltpu.VMEM((CH, D), jnp.float32),
        pltpu.VMEM((CH, D), jnp.float32),
        pltpu.VMEM_SHARED((NP, D), jnp.float32),
        pltpu.SemaphoreType.DMA,
        pltpu.SemaphoreType.DMA,
        pltpu.SemaphoreType.DMA,
        pltpu.SemaphoreType.DMA,
    ],
)
def _sc_gs(xs_hbm, row_hbm, col_hbm, zeros2_hbm, gp,
           rows_v, cols_v, buf0, buf1, g_sh, sg0, sg1, ss0, ss1):
    c = lax.axis_index("c")
    s = lax.axis_index("s")
    w = s * NC + c
    pltpu.sync_copy(zeros2_hbm.at[pl.ds(s * RPS, RPS)],
                    g_sh.at[pl.ds(s * RPS, RPS)])
    plsc.subcore_barrier()

    for h in range(2):
        pltpu.sync_copy(row_hbm.at[pl.ds(w * KCH + h * HALF, HALF)], rows_v)
        pltpu.sync_copy(col_hbm.at[pl.ds(w * KCH + h * HALF, HALF)], cols_v)

        pltpu.async_copy(xs_hbm.at[cols_v.at[0]], buf0, sg0)
        pltpu.async_copy(xs_hbm.at[cols_v.at[1]], buf1, sg1)

        @pl.loop(0, HALF // 2 - 1)
        def _(j):
            k = j * 2
            pltpu.make_async_copy(xs_hbm.at[cols_v.at[k]], buf0, sg0).wait()
            d0 = pltpu.async_copy(buf0, g_sh.at[rows_v.at[k]], ss0, add=True)
            pltpu.make_async_copy(xs_hbm.at[cols_v.at[k + 1]],
                                  buf1, sg1).wait()
            d1 = pltpu.async_copy(buf1, g_sh.at[rows_v.at[k + 1]],
                                  ss1, add=True)
            d0.wait()
            pltpu.async_copy(xs_hbm.at[cols_v.at[k + 2]], buf0, sg0)
            d1.wait()
            pltpu.async_copy(xs_hbm.at[cols_v.at[k + 3]], buf1, sg1)

        last = HALF - 2
        pltpu.make_async_copy(xs_hbm.at[cols_v.at[last]], buf0, sg0).wait()
        d0 = pltpu.async_copy(buf0, g_sh.at[rows_v.at[last]], ss0, add=True)
        pltpu.make_async_copy(xs_hbm.at[cols_v.at[last + 1]], buf1, sg1).wait()
        d1 = pltpu.async_copy(buf1, g_sh.at[rows_v.at[last + 1]],
                              ss1, add=True)
        d0.wait()
        d1.wait()

    plsc.subcore_barrier()
    pltpu.sync_copy(g_sh.at[pl.ds(s * RPS, RPS)],
                    gp.at[c, pl.ds(s * RPS, RPS)])


_R = 1024


def _prep_body(d0, d1, s0, s1, dis_ref, u_ref):
    deg = d0[...] + d1[...] + 1.0
    cnt = s0[...] + s1[...]
    i = (lax.broadcasted_iota(jnp.int32, (NP // 128, 128), 0) * 128
         + lax.broadcasted_iota(jnp.int32, (NP // 128, 128), 1))
    mask = i < N
    dis_ref[...] = jnp.where(mask, lax.rsqrt(deg), 0.0)
    u_ref[...] = jnp.where(mask, (deg - 1.0) / deg + cnt, 0.0)


def _tc_prep(d0, d1, s0, s1):
    f = pl.pallas_call(
        _prep_body,
        out_shape=(jax.ShapeDtypeStruct((NP // 128, 128), jnp.float32),
                   jax.ShapeDtypeStruct((NP // 128, 128), jnp.float32)),
    )
    return f(d0, d1, s0, s1)


def _scale_body(x, dis, xs_ref):
    xs_ref[...] = x[...] * dis[...]


def _tc_scale(x, dis):
    f = pl.pallas_call(
        _scale_body,
        grid=(NP // _R,),
        in_specs=[
            pl.BlockSpec((_R, D), lambda i: (i, 0)),
            pl.BlockSpec((_R, 1), lambda i: (i, 0)),
        ],
        out_specs=pl.BlockSpec((_R, D), lambda i: (i, 0)),
        out_shape=jax.ShapeDtypeStruct((NP, D), jnp.float32),
    )
    return f(x, dis)


def _mid_body(x, g0, g1, dis, u, phi, y_ref, ys_ref):
    g = g0[...] + g1[...]
    sp = u[...] * x[...] - dis[...] * g
    y = x[...] - phi[...] * sp
    y_ref[...] = y
    ys_ref[...] = dis[...] * y


def _tc_mid(x, g0, g1, dis, u, phi):
    f = pl.pallas_call(
        _mid_body,
        grid=(NP // _R,),
        in_specs=[
            pl.BlockSpec((_R, D), lambda i: (i, 0)),
            pl.BlockSpec((_R, D), lambda i: (i, 0)),
            pl.BlockSpec((_R, D), lambda i: (i, 0)),
            pl.BlockSpec((_R, 1), lambda i: (i, 0)),
            pl.BlockSpec((_R, 1), lambda i: (i, 0)),
            pl.BlockSpec((1, D), lambda i: (0, 0)),
        ],
        out_specs=(pl.BlockSpec((_R, D), lambda i: (i, 0)),
                   pl.BlockSpec((_R, D), lambda i: (i, 0))),
        out_shape=(jax.ShapeDtypeStruct((NP, D), jnp.float32),
                   jax.ShapeDtypeStruct((NP, D), jnp.float32)),
    )
    return f(x, g0, g1, dis, u, phi)


def _mm_body(relu, x, g0, g1, dis, u, phi, W, b, y_ref, ys_ref=None):
    g = g0[...] + g1[...]
    z = x[...] - phi[...] * (u[...] * x[...] - dis[...] * g)
    y = jnp.dot(z, W[...], preferred_element_type=jnp.float32) + b[...]
    if relu:
        y = jnp.maximum(y, 0.0)
    y_ref[...] = y
    if ys_ref is not None:
        ys_ref[...] = dis[...] * y


def _tc_mm(x, g0, g1, dis, u, phi, W, b, relu, want_ys):
    nout = 2 if want_ys else 1
    blk = pl.BlockSpec((_R, D), lambda i: (i, 0))
    out_specs = (blk, blk) if want_ys else blk
    out_shape = tuple(jax.ShapeDtypeStruct((NP, D), jnp.float32)
                      for _ in range(nout))
    if not want_ys:
        out_shape = out_shape[0]
    f = pl.pallas_call(
        functools.partial(_mm_body, relu),
        grid=(NP // _R,),
        in_specs=[
            blk, blk, blk,
            pl.BlockSpec((_R, 1), lambda i: (i, 0)),
            pl.BlockSpec((_R, 1), lambda i: (i, 0)),
            pl.BlockSpec((1, D), lambda i: (0, 0)),
            pl.BlockSpec((D, D), lambda i: (0, 0)),
            pl.BlockSpec((1, D), lambda i: (0, 0)),
        ],
        out_specs=out_specs,
        out_shape=out_shape,
    )
    return f(x, g0, g1, dis, u, phi, W, b)


@jax.jit
def _run(node_feat, edge_index, phi1, W1, b1, phi_hidden, phi2, W2, b2):
    xpad = jnp.pad(node_feat, ((0, NP - N), (0, 0)))
    rowp = jnp.pad(edge_index[0], (0, EP - E),
                   constant_values=NP - 1).reshape(NW * KCH, CH)
    colp = jnp.pad(edge_index[1], (0, EP - E),
                   constant_values=NP - 1).reshape(NW * KCH, CH)
    zeros1 = jnp.zeros((NP,), jnp.float32)
    zeros2 = jnp.zeros((NP, D), jnp.float32)

    degp, selfp = _sc_hist(rowp, colp, zeros1)
    dis2d, u2d = _tc_prep(degp[0].reshape(NP // 128, 128),
                          degp[1].reshape(NP // 128, 128),
                          selfp[0].reshape(NP // 128, 128),
                          selfp[1].reshape(NP // 128, 128))
    dis = dis2d.reshape(NP, 1)
    u = u2d.reshape(NP, 1)

    xs = _tc_scale(xpad, dis)
    g = _sc_gs(xs, rowp, colp, zeros2)
    x1, xs = _tc_mm(xpad, g[0], g[1], dis, u, phi1.reshape(1, D), W1,
                    b1.reshape(1, D), relu=True, want_ys=True)
    g = _sc_gs(xs, rowp, colp, zeros2)
    x2, xs = _tc_mid(x1, g[0], g[1], dis, u, phi_hidden[0].reshape(1, D))
    g = _sc_gs(xs, rowp, colp, zeros2)
    x3, xs = _tc_mid(x2, g[0], g[1], dis, u, phi_hidden[1].reshape(1, D))
    g = _sc_gs(xs, rowp, colp, zeros2)
    out = _tc_mm(x3, g[0], g[1], dis, u, phi2.reshape(1, D), W2,
                 b2.reshape(1, D), relu=False, want_ys=False)
    return out[:N]


def kernel(node_feat, edge_index, phi1, W1, b1, phi_hidden, phi2, W2, b2):
    return _run(node_feat, edge_index, phi1, W1, b1, phi_hidden, phi2, W2, b2)


# speedup vs baseline: 1.0003x; 1.0003x over previous
import functools

import jax
import jax.numpy as jnp
from jax import lax
from jax.experimental import pallas as pl
from jax.experimental.pallas import tpu as pltpu
from jax.experimental.pallas import tpu_sc as plsc

N = 10000
NP = 10240
D = 128
E = 320000
CH = 128
NC = 2
NS = 16
NW = NC * NS
KCH = 80
EP = NW * KCH * CH
RPS = NP // NS
FPT = D // NW
GCH = 1024
NCHK = EP // GCH

_mesh = plsc.VectorSubcoreMesh(core_axis_name="c", subcore_axis_name="s")


@functools.partial(
    pl.kernel,
    out_type=(
        jax.ShapeDtypeStruct((NC, NP), jnp.float32),
        jax.ShapeDtypeStruct((NC, NP), jnp.float32),
        jax.ShapeDtypeStruct((NW * KCH, CH), jnp.int32),
    ),
    mesh=_mesh,
    scratch_types=[
        pltpu.VMEM((KCH, CH), jnp.int32),
        pltpu.VMEM((KCH, CH), jnp.int32),
        pltpu.VMEM((KCH, CH), jnp.int32),
        pltpu.VMEM((CH,), jnp.float32),
        pltpu.VMEM((1, CH), jnp.int32),
        pltpu.VMEM_SHARED((NP,), jnp.float32),
        pltpu.VMEM_SHARED((NP,), jnp.float32),
    ],
)
def _sc_hist(row_hbm, col_hbm, zeros1_hbm, degp, selfp, packed,
             rows_v, cols_v, pk_v, ones_v, sel_v, deg_sh, self_sh):
    c = lax.axis_index("c")
    s = lax.axis_index("s")
    w = s * NC + c
    pltpu.sync_copy(zeros1_hbm.at[pl.ds(s * RPS, RPS)],
                    deg_sh.at[pl.ds(s * RPS, RPS)])
    pltpu.sync_copy(zeros1_hbm.at[pl.ds(s * RPS, RPS)],
                    self_sh.at[pl.ds(s * RPS, RPS)])
    for j in range(CH // 16):
        ones_v[pl.ds(j * 16, 16)] = jnp.ones((16,), jnp.float32)
    pltpu.sync_copy(row_hbm.at[pl.ds(w * KCH, KCH)], rows_v)
    pltpu.sync_copy(col_hbm.at[pl.ds(w * KCH, KCH)], cols_v)
    plsc.subcore_barrier()

    @pl.loop(0, KCH)
    def _(k):
        pltpu.sync_copy(ones_v, deg_sh.at[cols_v.at[k]], add=True)
        for j in range(CH // 16):
            r = rows_v[k, pl.ds(j * 16, 16)]
            cc = cols_v[k, pl.ds(j * 16, 16)]
            sel_v[0, pl.ds(j * 16, 16)] = jnp.where(r == cc, cc, NP - 1)
            pk_v[k, pl.ds(j * 16, 16)] = r * 16384 + cc
        pltpu.sync_copy(ones_v, self_sh.at[sel_v.at[0]], add=True)

    pltpu.sync_copy(pk_v, packed.at[pl.ds(w * KCH, KCH)])
    plsc.subcore_barrier()
    pltpu.sync_copy(deg_sh.at[pl.ds(s * RPS, RPS)],
                    degp.at[c, pl.ds(s * RPS, RPS)])
    pltpu.sync_copy(self_sh.at[pl.ds(s * RPS, RPS)],
                    selfp.at[c, pl.ds(s * RPS, RPS)])


@functools.partial(
    pl.kernel,
    out_type=jax.ShapeDtypeStruct((NW, FPT * NP), jnp.float32),
    mesh=_mesh,
    scratch_types=[
        pltpu.VMEM((FPT * NP,), jnp.float32),
        pltpu.VMEM((FPT * NP,), jnp.float32),
        pltpu.VMEM((GCH,), jnp.int32),
        pltpu.VMEM((GCH,), jnp.int32),
        pltpu.SemaphoreType.DMA,
        pltpu.SemaphoreType.DMA,
        pltpu.SemaphoreType.DMA,
    ],
    compiler_params=pltpu.CompilerParams(needs_layout_passes=False),
)
def _sc_gs(xs_hbm, pk_hbm, gp, xs_v, acc_v, pb0, pb1, sx, s0, s1):
    c = lax.axis_index("c")
    s = lax.axis_index("s")
    w = s * NC + c
    dx = pltpu.async_copy(xs_hbm.at[w], xs_v, sx)
    d0 = pltpu.async_copy(pk_hbm.at[pl.ds(0, GCH)], pb0, s0)

    @pl.loop(0, NP // 16)
    def _(i):
        for f in range(FPT):
            acc_v[pl.ds(f * NP + i * 16, 16)] = jnp.zeros((16,), jnp.float32)

    dx.wait()

    def process(buf):
        @pl.loop(0, GCH // 16, unroll=16)
        def _(g):
            p = buf[pl.ds(g * 16, 16)]
            cc = jnp.bitwise_and(p, 16383)
            r = lax.shift_right_logical(p, 14)
            for f in range(FPT):
                v = plsc.load_gather(xs_v, [cc + f * NP])
                plsc.addupdate_scatter(acc_v, [r + f * NP], v)

    @pl.loop(0, NCHK // 2)
    def _(j):
        k = j * 2
        pltpu.make_async_copy(pk_hbm.at[pl.ds(0, GCH)], pb0, s0).wait()
        nxt = jnp.minimum((k + 1) * GCH, (NCHK - 1) * GCH)
        pltpu.async_copy(pk_hbm.at[pl.ds(nxt, GCH)], pb1, s1)
        process(pb0)
        pltpu.make_async_copy(pk_hbm.at[pl.ds(0, GCH)], pb1, s1).wait()
        nxt2 = jnp.minimum((k + 2) * GCH, (NCHK - 1) * GCH)
        pltpu.async_copy(pk_hbm.at[pl.ds(nxt2, GCH)], pb0, s0)
        process(pb1)

    pltpu.make_async_copy(pk_hbm.at[pl.ds(0, GCH)], pb0, s0).wait()
    pltpu.sync_copy(acc_v, gp.at[w])


_L = 2048


def _prep_body(d0, d1, s0, s1, dis_ref, u_ref):
    deg = d0[...] + d1[...] + 1.0
    cnt = s0[...] + s1[...]
    mask = lax.broadcasted_iota(jnp.int32, (1, NP), 1) < N
    dis_ref[...] = jnp.where(mask, lax.rsqrt(deg), 0.0)
    u_ref[...] = jnp.where(mask, (deg - 1.0) / deg + cnt, 0.0)


def _tc_prep(d0, d1, s0, s1):
    f = pl.pallas_call(
        _prep_body,
        out_shape=(jax.ShapeDtypeStruct((1, NP), jnp.float32),
                   jax.ShapeDtypeStruct((1, NP), jnp.float32)),
    )
    return f(d0, d1, s0, s1)


def _scale_body(x, dis, xs_ref):
    xs_ref[...] = x[...] * dis[...]


def _tc_scale(x, dis):
    f = pl.pallas_call(
        _scale_body,
        grid=(NP // _L,),
        in_specs=[
            pl.BlockSpec((D, _L), lambda i: (0, i)),
            pl.BlockSpec((1, _L), lambda i: (0, i)),
        ],
        out_specs=pl.BlockSpec((D, _L), lambda i: (0, i)),
        out_shape=jax.ShapeDtypeStruct((D, NP), jnp.float32),
    )
    return f(x, dis)


def _mid_body(x, g, dis, u, phi, y_ref, ys_ref):
    sp = u[...] * x[...] - dis[...] * g[...]
    y = x[...] - phi[...] * sp
    y_ref[...] = y
    ys_ref[...] = dis[...] * y


def _tc_mid(x, g, dis, u, phi):
    f = pl.pallas_call(
        _mid_body,
        grid=(NP // _L,),
        in_specs=[
            pl.BlockSpec((D, _L), lambda i: (0, i)),
            pl.BlockSpec((D, _L), lambda i: (0, i)),
            pl.BlockSpec((1, _L), lambda i: (0, i)),
            pl.BlockSpec((1, _L), lambda i: (0, i)),
            pl.BlockSpec((D, 1), lambda i: (0, 0)),
        ],
        out_specs=(pl.BlockSpec((D, _L), lambda i: (0, i)),
                   pl.BlockSpec((D, _L), lambda i: (0, i))),
        out_shape=(jax.ShapeDtypeStruct((D, NP), jnp.float32),
                   jax.ShapeDtypeStruct((D, NP), jnp.float32)),
    )
    return f(x, g, dis, u, phi)


def _mm_body(relu, x, g, dis, u, phi, Wt, b, y_ref, ys_ref=None):
    z = x[...] - phi[...] * (u[...] * x[...] - dis[...] * g[...])
    y = jnp.dot(Wt[...], z, preferred_element_type=jnp.float32) + b[...]
    if relu:
        y = jnp.maximum(y, 0.0)
    y_ref[...] = y
    if ys_ref is not None:
        ys_ref[...] = dis[...] * y


def _tc_mm(x, g, dis, u, phi, Wt, b, relu, want_ys):
    blk = pl.BlockSpec((D, _L), lambda i: (0, i))
    out_specs = (blk, blk) if want_ys else blk
    out_shape = jax.ShapeDtypeStruct((D, NP), jnp.float32)
    if want_ys:
        out_shape = (out_shape, out_shape)
    f = pl.pallas_call(
        functools.partial(_mm_body, relu),
        grid=(NP // _L,),
        in_specs=[
            blk, blk,
            pl.BlockSpec((1, _L), lambda i: (0, i)),
            pl.BlockSpec((1, _L), lambda i: (0, i)),
            pl.BlockSpec((D, 1), lambda i: (0, 0)),
            pl.BlockSpec((D, D), lambda i: (0, 0)),
            pl.BlockSpec((D, 1), lambda i: (0, 0)),
        ],
        out_specs=out_specs,
        out_shape=out_shape,
    )
    return f(x, g, dis, u, phi, Wt, b)


@jax.jit
def _run(node_feat, edge_index, phi1, W1, b1, phi_hidden, phi2, W2, b2):
    xt = jnp.pad(node_feat, ((0, NP - N), (0, 0))).T
    rowp = jnp.pad(edge_index[0], (0, EP - E),
                   constant_values=NP - 1).reshape(NW * KCH, CH)
    colp = jnp.pad(edge_index[1], (0, EP - E),
                   constant_values=NP - 1).reshape(NW * KCH, CH)
    zeros1 = jnp.zeros((NP,), jnp.float32)

    degp, selfp, packed = _sc_hist(rowp, colp, zeros1)
    pk = packed.reshape(EP)
    dis, u = _tc_prep(degp[0].reshape(1, NP), degp[1].reshape(1, NP),
                      selfp[0].reshape(1, NP), selfp[1].reshape(1, NP))

    xs = _tc_scale(xt, dis)
    g = _sc_gs(xs.reshape(NW, FPT * NP), pk).reshape(D, NP)
    x1, xs = _tc_mm(xt, g, dis, u, phi1.reshape(D, 1), W1.T,
                    b1.reshape(D, 1), relu=True, want_ys=True)
    g = _sc_gs(xs.reshape(NW, FPT * NP), pk).reshape(D, NP)
    x2, xs = _tc_mid(x1, g, dis, u, phi_hidden[0].reshape(D, 1))
    g = _sc_gs(xs.reshape(NW, FPT * NP), pk).reshape(D, NP)
    x3, xs = _tc_mid(x2, g, dis, u, phi_hidden[1].reshape(D, 1))
    g = _sc_gs(xs.reshape(NW, FPT * NP), pk).reshape(D, NP)
    out = _tc_mm(x3, g, dis, u, phi2.reshape(D, 1), W2.T,
                 b2.reshape(D, 1), relu=False, want_ys=False)
    return out[:, :N].T


def kernel(node_feat, edge_index, phi1, W1, b1, phi_hidden, phi2, W2, b2):
    return _run(node_feat, edge_index, phi1, W1, b1, phi_hidden, phi2, W2, b2)


# speedup vs baseline: 1.1528x; 1.1525x over previous
import functools

import jax
import jax.numpy as jnp
from jax import lax
from jax.experimental import pallas as pl
from jax.experimental.pallas import tpu as pltpu
from jax.experimental.pallas import tpu_sc as plsc

N = 10000
NP = 10240
D = 128
E = 320000
CH = 128
NC = 2
NS = 16
NW = NC * NS
KCH = 80
EP = NW * KCH * CH
RPS = NP // NS
FPT = D // NW
GCH = 1024
NCHK = EP // GCH

_mesh = plsc.VectorSubcoreMesh(core_axis_name="c", subcore_axis_name="s")


@functools.partial(
    pl.kernel,
    out_type=(
        jax.ShapeDtypeStruct((NC, NP), jnp.float32),
        jax.ShapeDtypeStruct((NC, NP), jnp.float32),
        jax.ShapeDtypeStruct((NW * KCH, CH), jnp.int32),
    ),
    mesh=_mesh,
    scratch_types=[
        pltpu.VMEM((KCH, CH), jnp.int32),
        pltpu.VMEM((KCH, CH), jnp.int32),
        pltpu.VMEM((KCH, CH), jnp.int32),
        pltpu.VMEM((CH,), jnp.float32),
        pltpu.VMEM((1, CH), jnp.int32),
        pltpu.VMEM_SHARED((NP,), jnp.float32),
        pltpu.VMEM_SHARED((NP,), jnp.float32),
    ],
)
def _sc_hist(row_hbm, col_hbm, zeros1_hbm, degp, selfp, packed,
             rows_v, cols_v, pk_v, ones_v, sel_v, deg_sh, self_sh):
    c = lax.axis_index("c")
    s = lax.axis_index("s")
    w = s * NC + c
    pltpu.sync_copy(zeros1_hbm.at[pl.ds(s * RPS, RPS)],
                    deg_sh.at[pl.ds(s * RPS, RPS)])
    pltpu.sync_copy(zeros1_hbm.at[pl.ds(s * RPS, RPS)],
                    self_sh.at[pl.ds(s * RPS, RPS)])
    for j in range(CH // 16):
        ones_v[pl.ds(j * 16, 16)] = jnp.ones((16,), jnp.float32)
    pltpu.sync_copy(row_hbm.at[pl.ds(w * KCH, KCH)], rows_v)
    pltpu.sync_copy(col_hbm.at[pl.ds(w * KCH, KCH)], cols_v)
    plsc.subcore_barrier()

    @pl.loop(0, KCH)
    def _(k):
        pltpu.sync_copy(ones_v, deg_sh.at[cols_v.at[k]], add=True)
        for j in range(CH // 16):
            r = rows_v[k, pl.ds(j * 16, 16)]
            cc = cols_v[k, pl.ds(j * 16, 16)]
            sel_v[0, pl.ds(j * 16, 16)] = jnp.where(r == cc, cc, NP - 1)
            pk_v[k, pl.ds(j * 16, 16)] = r * 16384 + cc
        pltpu.sync_copy(ones_v, self_sh.at[sel_v.at[0]], add=True)

    pltpu.sync_copy(pk_v, packed.at[pl.ds(w * KCH, KCH)])
    plsc.subcore_barrier()
    pltpu.sync_copy(deg_sh.at[pl.ds(s * RPS, RPS)],
                    degp.at[c, pl.ds(s * RPS, RPS)])
    pltpu.sync_copy(self_sh.at[pl.ds(s * RPS, RPS)],
                    selfp.at[c, pl.ds(s * RPS, RPS)])


@functools.partial(
    pl.kernel,
    out_type=jax.ShapeDtypeStruct((NW, FPT * NP), jnp.float32),
    mesh=_mesh,
    scratch_types=[
        pltpu.VMEM((FPT * NP,), jnp.float32),
        pltpu.VMEM((FPT * NP,), jnp.float32),
        pltpu.VMEM((GCH,), jnp.int32),
        pltpu.VMEM((GCH,), jnp.int32),
        pltpu.SemaphoreType.DMA,
        pltpu.SemaphoreType.DMA,
        pltpu.SemaphoreType.DMA,
    ],
    compiler_params=pltpu.CompilerParams(needs_layout_passes=False),
)
def _sc_gs(xs_hbm, pk_hbm, gp, xs_v, acc_v, pb0, pb1, sx, s0, s1):
    c = lax.axis_index("c")
    s = lax.axis_index("s")
    w = s * NC + c
    dx = pltpu.async_copy(xs_hbm.at[w], xs_v, sx)
    d0 = pltpu.async_copy(pk_hbm.at[pl.ds(0, GCH)], pb0, s0)

    @pl.loop(0, NP // 16)
    def _(i):
        for f in range(FPT):
            acc_v[pl.ds(f * NP + i * 16, 16)] = jnp.zeros((16,), jnp.float32)

    dx.wait()

    def process(buf):
        @pl.loop(0, GCH // 64, unroll=4)
        def _(q):
            ps = [buf[pl.ds(q * 64 + t * 16, 16)] for t in range(4)]
            ccs = [jnp.bitwise_and(p, 16383) for p in ps]
            rs = [lax.shift_right_logical(p, 14) for p in ps]
            for f in range(FPT):
                for t in range(4):
                    v = plsc.load_gather(xs_v, [ccs[t] + f * NP])
                    plsc.addupdate_scatter(acc_v, [rs[t] + f * NP], v)

    @pl.loop(0, NCHK // 2)
    def _(j):
        k = j * 2
        pltpu.make_async_copy(pk_hbm.at[pl.ds(0, GCH)], pb0, s0).wait()
        nxt = jnp.minimum((k + 1) * GCH, (NCHK - 1) * GCH)
        pltpu.async_copy(pk_hbm.at[pl.ds(nxt, GCH)], pb1, s1)
        process(pb0)
        pltpu.make_async_copy(pk_hbm.at[pl.ds(0, GCH)], pb1, s1).wait()
        nxt2 = jnp.minimum((k + 2) * GCH, (NCHK - 1) * GCH)
        pltpu.async_copy(pk_hbm.at[pl.ds(nxt2, GCH)], pb0, s0)
        process(pb1)

    pltpu.make_async_copy(pk_hbm.at[pl.ds(0, GCH)], pb0, s0).wait()
    pltpu.sync_copy(acc_v, gp.at[w])


_L = 2048


def _prep_body(d0, d1, s0, s1, dis_ref, u_ref):
    deg = d0[...] + d1[...] + 1.0
    cnt = s0[...] + s1[...]
    mask = lax.broadcasted_iota(jnp.int32, (1, NP), 1) < N
    dis_ref[...] = jnp.where(mask, lax.rsqrt(deg), 0.0)
    u_ref[...] = jnp.where(mask, (deg - 1.0) / deg + cnt, 0.0)


def _tc_prep(d0, d1, s0, s1):
    f = pl.pallas_call(
        _prep_body,
        out_shape=(jax.ShapeDtypeStruct((1, NP), jnp.float32),
                   jax.ShapeDtypeStruct((1, NP), jnp.float32)),
    )
    return f(d0, d1, s0, s1)


def _scale_body(x, dis, xs_ref):
    xs_ref[...] = x[...] * dis[...]


def _tc_scale(x, dis):
    f = pl.pallas_call(
        _scale_body,
        grid=(NP // _L,),
        in_specs=[
            pl.BlockSpec((D, _L), lambda i: (0, i)),
            pl.BlockSpec((1, _L), lambda i: (0, i)),
        ],
        out_specs=pl.BlockSpec((D, _L), lambda i: (0, i)),
        out_shape=jax.ShapeDtypeStruct((D, NP), jnp.float32),
    )
    return f(x, dis)


def _mid_body(x, g, dis, u, phi, y_ref, ys_ref):
    sp = u[...] * x[...] - dis[...] * g[...]
    y = x[...] - phi[...] * sp
    y_ref[...] = y
    ys_ref[...] = dis[...] * y


def _tc_mid(x, g, dis, u, phi):
    f = pl.pallas_call(
        _mid_body,
        grid=(NP // _L,),
        in_specs=[
            pl.BlockSpec((D, _L), lambda i: (0, i)),
            pl.BlockSpec((D, _L), lambda i: (0, i)),
            pl.BlockSpec((1, _L), lambda i: (0, i)),
            pl.BlockSpec((1, _L), lambda i: (0, i)),
            pl.BlockSpec((D, 1), lambda i: (0, 0)),
        ],
        out_specs=(pl.BlockSpec((D, _L), lambda i: (0, i)),
                   pl.BlockSpec((D, _L), lambda i: (0, i))),
        out_shape=(jax.ShapeDtypeStruct((D, NP), jnp.float32),
                   jax.ShapeDtypeStruct((D, NP), jnp.float32)),
    )
    return f(x, g, dis, u, phi)


def _mm_body(relu, x, g, dis, u, phi, Wt, b, y_ref, ys_ref=None):
    z = x[...] - phi[...] * (u[...] * x[...] - dis[...] * g[...])
    y = jnp.dot(Wt[...], z, preferred_element_type=jnp.float32) + b[...]
    if relu:
        y = jnp.maximum(y, 0.0)
    y_ref[...] = y
    if ys_ref is not None:
        ys_ref[...] = dis[...] * y


def _tc_mm(x, g, dis, u, phi, Wt, b, relu, want_ys):
    blk = pl.BlockSpec((D, _L), lambda i: (0, i))
    out_specs = (blk, blk) if want_ys else blk
    out_shape = jax.ShapeDtypeStruct((D, NP), jnp.float32)
    if want_ys:
        out_shape = (out_shape, out_shape)
    f = pl.pallas_call(
        functools.partial(_mm_body, relu),
        grid=(NP // _L,),
        in_specs=[
            blk, blk,
            pl.BlockSpec((1, _L), lambda i: (0, i)),
            pl.BlockSpec((1, _L), lambda i: (0, i)),
            pl.BlockSpec((D, 1), lambda i: (0, 0)),
            pl.BlockSpec((D, D), lambda i: (0, 0)),
            pl.BlockSpec((D, 1), lambda i: (0, 0)),
        ],
        out_specs=out_specs,
        out_shape=out_shape,
    )
    return f(x, g, dis, u, phi, Wt, b)


@jax.jit
def _run(node_feat, edge_index, phi1, W1, b1, phi_hidden, phi2, W2, b2):
    xt = jnp.pad(node_feat, ((0, NP - N), (0, 0))).T
    rowp = jnp.pad(edge_index[0], (0, EP - E),
                   constant_values=NP - 1).reshape(NW * KCH, CH)
    colp = jnp.pad(edge_index[1], (0, EP - E),
                   constant_values=NP - 1).reshape(NW * KCH, CH)
    zeros1 = jnp.zeros((NP,), jnp.float32)

    degp, selfp, packed = _sc_hist(rowp, colp, zeros1)
    pk = packed.reshape(EP)
    dis, u = _tc_prep(degp[0].reshape(1, NP), degp[1].reshape(1, NP),
                      selfp[0].reshape(1, NP), selfp[1].reshape(1, NP))

    xs = _tc_scale(xt, dis)
    g = _sc_gs(xs.reshape(NW, FPT * NP), pk).reshape(D, NP)
    x1, xs = _tc_mm(xt, g, dis, u, phi1.reshape(D, 1), W1.T,
                    b1.reshape(D, 1), relu=True, want_ys=True)
    g = _sc_gs(xs.reshape(NW, FPT * NP), pk).reshape(D, NP)
    x2, xs = _tc_mid(x1, g, dis, u, phi_hidden[0].reshape(D, 1))
    g = _sc_gs(xs.reshape(NW, FPT * NP), pk).reshape(D, NP)
    x3, xs = _tc_mid(x2, g, dis, u, phi_hidden[1].reshape(D, 1))
    g = _sc_gs(xs.reshape(NW, FPT * NP), pk).reshape(D, NP)
    out = _tc_mm(x3, g, dis, u, phi2.reshape(D, 1), W2.T,
                 b2.reshape(D, 1), relu=False, want_ys=False)
    return out[:, :N].T


def kernel(node_feat, edge_index, phi1, W1, b1, phi_hidden, phi2, W2, b2):
    return _run(node_feat, edge_index, phi1, W1, b1, phi_hidden, phi2, W2, b2)


# speedup vs baseline: 1.1858x; 1.0286x over previous
import functools

import jax
import jax.numpy as jnp
from jax import lax
from jax.experimental import pallas as pl
from jax.experimental.pallas import tpu as pltpu
from jax.experimental.pallas import tpu_sc as plsc

N = 10000
NP = 10240
D = 128
E = 320000
CH = 128
NC = 2
NS = 16
NW = NC * NS
KCH = 80
EP = NW * KCH * CH
RPS = NP // NS
FPT = D // NW
GCH = 1024
NCHK = EP // GCH

_mesh = plsc.VectorSubcoreMesh(core_axis_name="c", subcore_axis_name="s")


@functools.partial(
    pl.kernel,
    out_type=(
        jax.ShapeDtypeStruct((NC, NP), jnp.float32),
        jax.ShapeDtypeStruct((NC, NP), jnp.float32),
        jax.ShapeDtypeStruct((NW * KCH, CH), jnp.int32),
    ),
    mesh=_mesh,
    scratch_types=[
        pltpu.VMEM((KCH, CH), jnp.int32),
        pltpu.VMEM((KCH, CH), jnp.int32),
        pltpu.VMEM((KCH, CH), jnp.int32),
        pltpu.VMEM((CH,), jnp.float32),
        pltpu.VMEM((1, CH), jnp.int32),
        pltpu.VMEM_SHARED((NP,), jnp.float32),
        pltpu.VMEM_SHARED((NP,), jnp.float32),
    ],
)
def _sc_hist(row_hbm, col_hbm, zeros1_hbm, degp, selfp, packed,
             rows_v, cols_v, pk_v, ones_v, sel_v, deg_sh, self_sh):
    c = lax.axis_index("c")
    s = lax.axis_index("s")
    w = s * NC + c
    pltpu.sync_copy(zeros1_hbm.at[pl.ds(s * RPS, RPS)],
                    deg_sh.at[pl.ds(s * RPS, RPS)])
    pltpu.sync_copy(zeros1_hbm.at[pl.ds(s * RPS, RPS)],
                    self_sh.at[pl.ds(s * RPS, RPS)])
    for j in range(CH // 16):
        ones_v[pl.ds(j * 16, 16)] = jnp.ones((16,), jnp.float32)
    pltpu.sync_copy(row_hbm.at[pl.ds(w * KCH, KCH)], rows_v)
    pltpu.sync_copy(col_hbm.at[pl.ds(w * KCH, KCH)], cols_v)
    plsc.subcore_barrier()

    @pl.loop(0, KCH)
    def _(k):
        pltpu.sync_copy(ones_v, deg_sh.at[cols_v.at[k]], add=True)
        for j in range(CH // 16):
            r = rows_v[k, pl.ds(j * 16, 16)]
            cc = cols_v[k, pl.ds(j * 16, 16)]
            sel_v[0, pl.ds(j * 16, 16)] = jnp.where(r == cc, cc, NP - 1)
            pk_v[k, pl.ds(j * 16, 16)] = r * 16384 + cc
        pltpu.sync_copy(ones_v, self_sh.at[sel_v.at[0]], add=True)

    pltpu.sync_copy(pk_v, packed.at[pl.ds(w * KCH, KCH)])
    plsc.subcore_barrier()
    pltpu.sync_copy(deg_sh.at[pl.ds(s * RPS, RPS)],
                    degp.at[c, pl.ds(s * RPS, RPS)])
    pltpu.sync_copy(self_sh.at[pl.ds(s * RPS, RPS)],
                    selfp.at[c, pl.ds(s * RPS, RPS)])


@functools.partial(
    pl.kernel,
    out_type=jax.ShapeDtypeStruct((NW, FPT * NP), jnp.float32),
    mesh=_mesh,
    scratch_types=[
        pltpu.VMEM((FPT * NP,), jnp.float32),
        pltpu.VMEM((FPT * NP,), jnp.float32),
        pltpu.VMEM((GCH,), jnp.int32),
        pltpu.VMEM((GCH,), jnp.int32),
        pltpu.SemaphoreType.DMA,
        pltpu.SemaphoreType.DMA,
        pltpu.SemaphoreType.DMA,
    ],
    compiler_params=pltpu.CompilerParams(needs_layout_passes=False),
)
def _sc_gs(xs_hbm, pk_hbm, gp, xs_v, acc_v, pb0, pb1, sx, s0, s1):
    c = lax.axis_index("c")
    s = lax.axis_index("s")
    w = s * NC + c
    dx = pltpu.async_copy(xs_hbm.at[w], xs_v, sx)
    d0 = pltpu.async_copy(pk_hbm.at[pl.ds(0, GCH)], pb0, s0)

    @pl.loop(0, NP // 16)
    def _(i):
        for f in range(FPT):
            acc_v[pl.ds(f * NP + i * 16, 16)] = jnp.zeros((16,), jnp.float32)

    dx.wait()

    def process(buf):
        @pl.loop(0, GCH // 128, unroll=2)
        def _(q):
            ps = [buf[pl.ds(q * 128 + t * 16, 16)] for t in range(8)]
            ccs = [jnp.bitwise_and(p, 16383) for p in ps]
            rs = [lax.shift_right_logical(p, 14) for p in ps]
            for f in range(FPT):
                for t in range(8):
                    v = plsc.load_gather(xs_v, [ccs[t] + f * NP])
                    plsc.addupdate_scatter(acc_v, [rs[t] + f * NP], v)

    @pl.loop(0, NCHK // 2)
    def _(j):
        k = j * 2
        pltpu.make_async_copy(pk_hbm.at[pl.ds(0, GCH)], pb0, s0).wait()
        nxt = jnp.minimum((k + 1) * GCH, (NCHK - 1) * GCH)
        pltpu.async_copy(pk_hbm.at[pl.ds(nxt, GCH)], pb1, s1)
        process(pb0)
        pltpu.make_async_copy(pk_hbm.at[pl.ds(0, GCH)], pb1, s1).wait()
        nxt2 = jnp.minimum((k + 2) * GCH, (NCHK - 1) * GCH)
        pltpu.async_copy(pk_hbm.at[pl.ds(nxt2, GCH)], pb0, s0)
        process(pb1)

    pltpu.make_async_copy(pk_hbm.at[pl.ds(0, GCH)], pb0, s0).wait()
    pltpu.sync_copy(acc_v, gp.at[w])


_L = 2048


def _prep_body(d0, d1, s0, s1, dis_ref, u_ref):
    deg = d0[...] + d1[...] + 1.0
    cnt = s0[...] + s1[...]
    mask = lax.broadcasted_iota(jnp.int32, (1, NP), 1) < N
    dis_ref[...] = jnp.where(mask, lax.rsqrt(deg), 0.0)
    u_ref[...] = jnp.where(mask, (deg - 1.0) / deg + cnt, 0.0)


def _tc_prep(d0, d1, s0, s1):
    f = pl.pallas_call(
        _prep_body,
        out_shape=(jax.ShapeDtypeStruct((1, NP), jnp.float32),
                   jax.ShapeDtypeStruct((1, NP), jnp.float32)),
    )
    return f(d0, d1, s0, s1)


def _scale_body(x, dis, xs_ref):
    xs_ref[...] = x[...] * dis[...]


def _tc_scale(x, dis):
    f = pl.pallas_call(
        _scale_body,
        grid=(NP // _L,),
        in_specs=[
            pl.BlockSpec((D, _L), lambda i: (0, i)),
            pl.BlockSpec((1, _L), lambda i: (0, i)),
        ],
        out_specs=pl.BlockSpec((D, _L), lambda i: (0, i)),
        out_shape=jax.ShapeDtypeStruct((D, NP), jnp.float32),
    )
    return f(x, dis)


def _mid_body(x, g, dis, u, phi, y_ref, ys_ref):
    sp = u[...] * x[...] - dis[...] * g[...]
    y = x[...] - phi[...] * sp
    y_ref[...] = y
    ys_ref[...] = dis[...] * y


def _tc_mid(x, g, dis, u, phi):
    f = pl.pallas_call(
        _mid_body,
        grid=(NP // _L,),
        in_specs=[
            pl.BlockSpec((D, _L), lambda i: (0, i)),
            pl.BlockSpec((D, _L), lambda i: (0, i)),
            pl.BlockSpec((1, _L), lambda i: (0, i)),
            pl.BlockSpec((1, _L), lambda i: (0, i)),
            pl.BlockSpec((D, 1), lambda i: (0, 0)),
        ],
        out_specs=(pl.BlockSpec((D, _L), lambda i: (0, i)),
                   pl.BlockSpec((D, _L), lambda i: (0, i))),
        out_shape=(jax.ShapeDtypeStruct((D, NP), jnp.float32),
                   jax.ShapeDtypeStruct((D, NP), jnp.float32)),
    )
    return f(x, g, dis, u, phi)


def _mm_body(relu, x, g, dis, u, phi, Wt, b, y_ref, ys_ref=None):
    z = x[...] - phi[...] * (u[...] * x[...] - dis[...] * g[...])
    y = jnp.dot(Wt[...], z, preferred_element_type=jnp.float32) + b[...]
    if relu:
        y = jnp.maximum(y, 0.0)
    y_ref[...] = y
    if ys_ref is not None:
        ys_ref[...] = dis[...] * y


def _tc_mm(x, g, dis, u, phi, Wt, b, relu, want_ys):
    blk = pl.BlockSpec((D, _L), lambda i: (0, i))
    out_specs = (blk, blk) if want_ys else blk
    out_shape = jax.ShapeDtypeStruct((D, NP), jnp.float32)
    if want_ys:
        out_shape = (out_shape, out_shape)
    f = pl.pallas_call(
        functools.partial(_mm_body, relu),
        grid=(NP // _L,),
        in_specs=[
            blk, blk,
            pl.BlockSpec((1, _L), lambda i: (0, i)),
            pl.BlockSpec((1, _L), lambda i: (0, i)),
            pl.BlockSpec((D, 1), lambda i: (0, 0)),
            pl.BlockSpec((D, D), lambda i: (0, 0)),
            pl.BlockSpec((D, 1), lambda i: (0, 0)),
        ],
        out_specs=out_specs,
        out_shape=out_shape,
    )
    return f(x, g, dis, u, phi, Wt, b)


@jax.jit
def _run(node_feat, edge_index, phi1, W1, b1, phi_hidden, phi2, W2, b2):
    xt = jnp.pad(node_feat, ((0, NP - N), (0, 0))).T
    rowp = jnp.pad(edge_index[0], (0, EP - E),
                   constant_values=NP - 1).reshape(NW * KCH, CH)
    colp = jnp.pad(edge_index[1], (0, EP - E),
                   constant_values=NP - 1).reshape(NW * KCH, CH)
    zeros1 = jnp.zeros((NP,), jnp.float32)

    degp, selfp, packed = _sc_hist(rowp, colp, zeros1)
    pk = packed.reshape(EP)
    dis, u = _tc_prep(degp[0].reshape(1, NP), degp[1].reshape(1, NP),
                      selfp[0].reshape(1, NP), selfp[1].reshape(1, NP))

    xs = _tc_scale(xt, dis)
    g = _sc_gs(xs.reshape(NW, FPT * NP), pk).reshape(D, NP)
    x1, xs = _tc_mm(xt, g, dis, u, phi1.reshape(D, 1), W1.T,
                    b1.reshape(D, 1), relu=True, want_ys=True)
    g = _sc_gs(xs.reshape(NW, FPT * NP), pk).reshape(D, NP)
    x2, xs = _tc_mid(x1, g, dis, u, phi_hidden[0].reshape(D, 1))
    g = _sc_gs(xs.reshape(NW, FPT * NP), pk).reshape(D, NP)
    x3, xs = _tc_mid(x2, g, dis, u, phi_hidden[1].reshape(D, 1))
    g = _sc_gs(xs.reshape(NW, FPT * NP), pk).reshape(D, NP)
    out = _tc_mm(x3, g, dis, u, phi2.reshape(D, 1), W2.T,
                 b2.reshape(D, 1), relu=False, want_ys=False)
    return out[:, :N].T


def kernel(node_feat, edge_index, phi1, W1, b1, phi_hidden, phi2, W2, b2):
    return _run(node_feat, edge_index, phi1, W1, b1, phi_hidden, phi2, W2, b2)


# speedup vs baseline: 1.1870x; 1.0010x over previous
import functools

import jax
import jax.numpy as jnp
from jax import lax
from jax.experimental import pallas as pl
from jax.experimental.pallas import tpu as pltpu
from jax.experimental.pallas import tpu_sc as plsc

N = 10000
NP = 10240
D = 128
E = 320000
CH = 128
NC = 2
NS = 16
NW = NC * NS
KCH = 80
EP = NW * KCH * CH
RPS = NP // NS
FPT = D // NW
GCH = 1024
NCHK = EP // GCH

_mesh = plsc.VectorSubcoreMesh(core_axis_name="c", subcore_axis_name="s")


@functools.partial(
    pl.kernel,
    out_type=(
        jax.ShapeDtypeStruct((NC, NP), jnp.float32),
        jax.ShapeDtypeStruct((NC, NP), jnp.float32),
        jax.ShapeDtypeStruct((NW * KCH, CH), jnp.int32),
    ),
    mesh=_mesh,
    scratch_types=[
        pltpu.VMEM((KCH, CH), jnp.int32),
        pltpu.VMEM((KCH, CH), jnp.int32),
        pltpu.VMEM((KCH, CH), jnp.int32),
        pltpu.VMEM((CH,), jnp.float32),
        pltpu.VMEM((1, CH), jnp.int32),
        pltpu.VMEM_SHARED((NP,), jnp.float32),
        pltpu.VMEM_SHARED((NP,), jnp.float32),
    ],
)
def _sc_hist(row_hbm, col_hbm, zeros1_hbm, degp, selfp, packed,
             rows_v, cols_v, pk_v, ones_v, sel_v, deg_sh, self_sh):
    c = lax.axis_index("c")
    s = lax.axis_index("s")
    w = s * NC + c
    pltpu.sync_copy(zeros1_hbm.at[pl.ds(s * RPS, RPS)],
                    deg_sh.at[pl.ds(s * RPS, RPS)])
    pltpu.sync_copy(zeros1_hbm.at[pl.ds(s * RPS, RPS)],
                    self_sh.at[pl.ds(s * RPS, RPS)])
    for j in range(CH // 16):
        ones_v[pl.ds(j * 16, 16)] = jnp.ones((16,), jnp.float32)
    pltpu.sync_copy(row_hbm.at[pl.ds(w * KCH, KCH)], rows_v)
    pltpu.sync_copy(col_hbm.at[pl.ds(w * KCH, KCH)], cols_v)
    plsc.subcore_barrier()

    @pl.loop(0, KCH)
    def _(k):
        pltpu.sync_copy(ones_v, deg_sh.at[cols_v.at[k]], add=True)
        for j in range(CH // 16):
            r = rows_v[k, pl.ds(j * 16, 16)]
            cc = cols_v[k, pl.ds(j * 16, 16)]
            sel_v[0, pl.ds(j * 16, 16)] = jnp.where(r == cc, cc, NP - 1)
            pk_v[k, pl.ds(j * 16, 16)] = r * 16384 + cc
        pltpu.sync_copy(ones_v, self_sh.at[sel_v.at[0]], add=True)

    pltpu.sync_copy(pk_v, packed.at[pl.ds(w * KCH, KCH)])
    plsc.subcore_barrier()
    pltpu.sync_copy(deg_sh.at[pl.ds(s * RPS, RPS)],
                    degp.at[c, pl.ds(s * RPS, RPS)])
    pltpu.sync_copy(self_sh.at[pl.ds(s * RPS, RPS)],
                    selfp.at[c, pl.ds(s * RPS, RPS)])


@functools.partial(
    pl.kernel,
    out_type=jax.ShapeDtypeStruct((NW, FPT, NP), jnp.float32),
    mesh=_mesh,
    scratch_types=[
        [pltpu.VMEM((NP,), jnp.float32)] * FPT,
        [pltpu.VMEM((NP,), jnp.float32)] * FPT,
        pltpu.VMEM((GCH,), jnp.int32),
        pltpu.VMEM((GCH,), jnp.int32),
        pltpu.SemaphoreType.DMA,
        pltpu.SemaphoreType.DMA,
        pltpu.SemaphoreType.DMA,
    ],
    compiler_params=pltpu.CompilerParams(needs_layout_passes=False),
)
def _sc_gs(xs_hbm, pk_hbm, gp, xs_v, acc_v, pb0, pb1, sx, s0, s1):
    c = lax.axis_index("c")
    s = lax.axis_index("s")
    w = s * NC + c
    dxs = [pltpu.async_copy(xs_hbm.at[w, f], xs_v[f], sx)
           for f in range(FPT)]
    d0 = pltpu.async_copy(pk_hbm.at[pl.ds(0, GCH)], pb0, s0)

    @pl.loop(0, NP // 16)
    def _(i):
        for f in range(FPT):
            acc_v[f][pl.ds(i * 16, 16)] = jnp.zeros((16,), jnp.float32)

    for d in dxs:
        d.wait()

    def process(buf):
        @pl.loop(0, GCH // 128, unroll=2)
        def _(q):
            ps = [buf[pl.ds(q * 128 + t * 16, 16)] for t in range(8)]
            ccs = [jnp.bitwise_and(p, 16383) for p in ps]
            rs = [lax.shift_right_logical(p, 14) for p in ps]
            for f in range(FPT):
                for t in range(8):
                    v = plsc.load_gather(xs_v[f], [ccs[t]])
                    plsc.addupdate_scatter(acc_v[f], [rs[t]], v)

    @pl.loop(0, NCHK // 2)
    def _(j):
        k = j * 2
        pltpu.make_async_copy(pk_hbm.at[pl.ds(0, GCH)], pb0, s0).wait()
        nxt = jnp.minimum((k + 1) * GCH, (NCHK - 1) * GCH)
        pltpu.async_copy(pk_hbm.at[pl.ds(nxt, GCH)], pb1, s1)
        process(pb0)
        pltpu.make_async_copy(pk_hbm.at[pl.ds(0, GCH)], pb1, s1).wait()
        nxt2 = jnp.minimum((k + 2) * GCH, (NCHK - 1) * GCH)
        pltpu.async_copy(pk_hbm.at[pl.ds(nxt2, GCH)], pb0, s0)
        process(pb1)

    pltpu.make_async_copy(pk_hbm.at[pl.ds(0, GCH)], pb0, s0).wait()
    for f in range(FPT):
        pltpu.sync_copy(acc_v[f], gp.at[w, f])


_L = 2048


def _prep_body(d0, d1, s0, s1, dis_ref, u_ref):
    deg = d0[...] + d1[...] + 1.0
    cnt = s0[...] + s1[...]
    mask = lax.broadcasted_iota(jnp.int32, (1, NP), 1) < N
    dis_ref[...] = jnp.where(mask, lax.rsqrt(deg), 0.0)
    u_ref[...] = jnp.where(mask, (deg - 1.0) / deg + cnt, 0.0)


def _tc_prep(d0, d1, s0, s1):
    f = pl.pallas_call(
        _prep_body,
        out_shape=(jax.ShapeDtypeStruct((1, NP), jnp.float32),
                   jax.ShapeDtypeStruct((1, NP), jnp.float32)),
    )
    return f(d0, d1, s0, s1)


def _scale_body(x, dis, xs_ref):
    xs_ref[...] = x[...] * dis[...]


def _tc_scale(x, dis):
    f = pl.pallas_call(
        _scale_body,
        grid=(NP // _L,),
        in_specs=[
            pl.BlockSpec((D, _L), lambda i: (0, i)),
            pl.BlockSpec((1, _L), lambda i: (0, i)),
        ],
        out_specs=pl.BlockSpec((D, _L), lambda i: (0, i)),
        out_shape=jax.ShapeDtypeStruct((D, NP), jnp.float32),
    )
    return f(x, dis)


def _mid_body(x, g, dis, u, phi, y_ref, ys_ref):
    sp = u[...] * x[...] - dis[...] * g[...]
    y = x[...] - phi[...] * sp
    y_ref[...] = y
    ys_ref[...] = dis[...] * y


def _tc_mid(x, g, dis, u, phi):
    f = pl.pallas_call(
        _mid_body,
        grid=(NP // _L,),
        in_specs=[
            pl.BlockSpec((D, _L), lambda i: (0, i)),
            pl.BlockSpec((D, _L), lambda i: (0, i)),
            pl.BlockSpec((1, _L), lambda i: (0, i)),
            pl.BlockSpec((1, _L), lambda i: (0, i)),
            pl.BlockSpec((D, 1), lambda i: (0, 0)),
        ],
        out_specs=(pl.BlockSpec((D, _L), lambda i: (0, i)),
                   pl.BlockSpec((D, _L), lambda i: (0, i))),
        out_shape=(jax.ShapeDtypeStruct((D, NP), jnp.float32),
                   jax.ShapeDtypeStruct((D, NP), jnp.float32)),
    )
    return f(x, g, dis, u, phi)


def _mm_body(relu, x, g, dis, u, phi, Wt, b, y_ref, ys_ref=None):
    z = x[...] - phi[...] * (u[...] * x[...] - dis[...] * g[...])
    y = jnp.dot(Wt[...], z, preferred_element_type=jnp.float32) + b[...]
    if relu:
        y = jnp.maximum(y, 0.0)
    y_ref[...] = y
    if ys_ref is not None:
        ys_ref[...] = dis[...] * y


def _tc_mm(x, g, dis, u, phi, Wt, b, relu, want_ys):
    blk = pl.BlockSpec((D, _L), lambda i: (0, i))
    out_specs = (blk, blk) if want_ys else blk
    out_shape = jax.ShapeDtypeStruct((D, NP), jnp.float32)
    if want_ys:
        out_shape = (out_shape, out_shape)
    f = pl.pallas_call(
        functools.partial(_mm_body, relu),
        grid=(NP // _L,),
        in_specs=[
            blk, blk,
            pl.BlockSpec((1, _L), lambda i: (0, i)),
            pl.BlockSpec((1, _L), lambda i: (0, i)),
            pl.BlockSpec((D, 1), lambda i: (0, 0)),
            pl.BlockSpec((D, D), lambda i: (0, 0)),
            pl.BlockSpec((D, 1), lambda i: (0, 0)),
        ],
        out_specs=out_specs,
        out_shape=out_shape,
    )
    return f(x, g, dis, u, phi, Wt, b)


@jax.jit
def _run(node_feat, edge_index, phi1, W1, b1, phi_hidden, phi2, W2, b2):
    xt = jnp.pad(node_feat, ((0, NP - N), (0, 0))).T
    rowp = jnp.pad(edge_index[0], (0, EP - E),
                   constant_values=NP - 1).reshape(NW * KCH, CH)
    colp = jnp.pad(edge_index[1], (0, EP - E),
                   constant_values=NP - 1).reshape(NW * KCH, CH)
    zeros1 = jnp.zeros((NP,), jnp.float32)

    degp, selfp, packed = _sc_hist(rowp, colp, zeros1)
    pk = packed.reshape(EP)
    dis, u = _tc_prep(degp[0].reshape(1, NP), degp[1].reshape(1, NP),
                      selfp[0].reshape(1, NP), selfp[1].reshape(1, NP))

    xs = _tc_scale(xt, dis)
    g = _sc_gs(xs.reshape(NW, FPT, NP), pk).reshape(D, NP)
    x1, xs = _tc_mm(xt, g, dis, u, phi1.reshape(D, 1), W1.T,
                    b1.reshape(D, 1), relu=True, want_ys=True)
    g = _sc_gs(xs.reshape(NW, FPT, NP), pk).reshape(D, NP)
    x2, xs = _tc_mid(x1, g, dis, u, phi_hidden[0].reshape(D, 1))
    g = _sc_gs(xs.reshape(NW, FPT, NP), pk).reshape(D, NP)
    x3, xs = _tc_mid(x2, g, dis, u, phi_hidden[1].reshape(D, 1))
    g = _sc_gs(xs.reshape(NW, FPT, NP), pk).reshape(D, NP)
    out = _tc_mm(x3, g, dis, u, phi2.reshape(D, 1), W2.T,
                 b2.reshape(D, 1), relu=False, want_ys=False)
    return out[:, :N].T


def kernel(node_feat, edge_index, phi1, W1, b1, phi_hidden, phi2, W2, b2):
    return _run(node_feat, edge_index, phi1, W1, b1, phi_hidden, phi2, W2, b2)


# speedup vs baseline: 1.2844x; 1.0821x over previous
import functools

import jax
import jax.numpy as jnp
from jax import lax
from jax.experimental import pallas as pl
from jax.experimental.pallas import tpu as pltpu
from jax.experimental.pallas import tpu_sc as plsc

N = 10000
NP = 10240
D = 128
E = 320000
CH = 128
NC = 2
NS = 16
NW = NC * NS
KCH = 80
EP = NW * KCH * CH
RPS = NP // NS
FPT = D // NW
GCH = 1024
NCHK = EP // GCH

_mesh = plsc.VectorSubcoreMesh(core_axis_name="c", subcore_axis_name="s")


@functools.partial(
    pl.kernel,
    out_type=(
        jax.ShapeDtypeStruct((NW, NP), jnp.float32),
        jax.ShapeDtypeStruct((NW, NP), jnp.float32),
        jax.ShapeDtypeStruct((NW * KCH, CH), jnp.int32),
    ),
    mesh=_mesh,
    scratch_types=[
        pltpu.VMEM((KCH, CH), jnp.int32),
        pltpu.VMEM((KCH, CH), jnp.int32),
        pltpu.VMEM((KCH, CH), jnp.int32),
        pltpu.VMEM((NP,), jnp.float32),
        pltpu.VMEM((NP,), jnp.float32),
    ],
    compiler_params=pltpu.CompilerParams(needs_layout_passes=False),
)
def _sc_hist(row_hbm, col_hbm, degp, selfp, packed,
             rows_v, cols_v, pk_v, deg_b, self_b):
    c = lax.axis_index("c")
    s = lax.axis_index("s")
    w = s * NC + c
    pltpu.sync_copy(row_hbm.at[pl.ds(w * KCH, KCH)], rows_v)
    pltpu.sync_copy(col_hbm.at[pl.ds(w * KCH, KCH)], cols_v)

    @pl.loop(0, NP // 16)
    def _(i):
        deg_b[pl.ds(i * 16, 16)] = jnp.zeros((16,), jnp.float32)
        self_b[pl.ds(i * 16, 16)] = jnp.zeros((16,), jnp.float32)

    ones = jnp.ones((16,), jnp.float32)

    @pl.loop(0, KCH)
    def _(k):
        for j in range(CH // 16):
            r = rows_v[k, pl.ds(j * 16, 16)]
            cc = cols_v[k, pl.ds(j * 16, 16)]
            plsc.addupdate_scatter(deg_b, [cc], ones)
            sel = jnp.where(r == cc, cc, NP - 1)
            plsc.addupdate_scatter(self_b, [sel], ones)
            pk_v[k, pl.ds(j * 16, 16)] = r * 16384 + cc

    pltpu.sync_copy(pk_v, packed.at[pl.ds(w * KCH, KCH)])
    pltpu.sync_copy(deg_b, degp.at[w])
    pltpu.sync_copy(self_b, selfp.at[w])


@functools.partial(
    pl.kernel,
    out_type=jax.ShapeDtypeStruct((NW, FPT, NP), jnp.float32),
    mesh=_mesh,
    scratch_types=[
        [pltpu.VMEM((NP,), jnp.float32)] * FPT,
        [pltpu.VMEM((NP,), jnp.float32)] * FPT,
        pltpu.VMEM((GCH,), jnp.int32),
        pltpu.VMEM((GCH,), jnp.int32),
        pltpu.SemaphoreType.DMA,
        pltpu.SemaphoreType.DMA,
        pltpu.SemaphoreType.DMA,
    ],
    compiler_params=pltpu.CompilerParams(needs_layout_passes=False),
)
def _sc_gs(xs_hbm, pk_hbm, gp, xs_v, acc_v, pb0, pb1, sx, s0, s1):
    c = lax.axis_index("c")
    s = lax.axis_index("s")
    w = s * NC + c
    dxs = [pltpu.async_copy(xs_hbm.at[w, f], xs_v[f], sx)
           for f in range(FPT)]
    d0 = pltpu.async_copy(pk_hbm.at[pl.ds(0, GCH)], pb0, s0)

    @pl.loop(0, NP // 16)
    def _(i):
        for f in range(FPT):
            acc_v[f][pl.ds(i * 16, 16)] = jnp.zeros((16,), jnp.float32)

    for d in dxs:
        d.wait()

    def process(buf):
        @pl.loop(0, GCH // 128, unroll=2)
        def _(q):
            ps = [buf[pl.ds(q * 128 + t * 16, 16)] for t in range(8)]
            ccs = [jnp.bitwise_and(p, 16383) for p in ps]
            rs = [lax.shift_right_logical(p, 14) for p in ps]
            for f in range(FPT):
                for t in range(8):
                    v = plsc.load_gather(xs_v[f], [ccs[t]])
                    plsc.addupdate_scatter(acc_v[f], [rs[t]], v)

    @pl.loop(0, NCHK // 2)
    def _(j):
        k = j * 2
        pltpu.make_async_copy(pk_hbm.at[pl.ds(0, GCH)], pb0, s0).wait()
        nxt = jnp.minimum((k + 1) * GCH, (NCHK - 1) * GCH)
        pltpu.async_copy(pk_hbm.at[pl.ds(nxt, GCH)], pb1, s1)
        process(pb0)
        pltpu.make_async_copy(pk_hbm.at[pl.ds(0, GCH)], pb1, s1).wait()
        nxt2 = jnp.minimum((k + 2) * GCH, (NCHK - 1) * GCH)
        pltpu.async_copy(pk_hbm.at[pl.ds(nxt2, GCH)], pb0, s0)
        process(pb1)

    pltpu.make_async_copy(pk_hbm.at[pl.ds(0, GCH)], pb0, s0).wait()
    for f in range(FPT):
        pltpu.sync_copy(acc_v[f], gp.at[w, f])


_L = 2048


def _prep_body(dp, sp, dis_ref, u_ref):
    i = pl.program_id(0)
    deg = jnp.sum(dp[...], axis=0, keepdims=True) + 1.0
    cnt = jnp.sum(sp[...], axis=0, keepdims=True)
    mask = lax.broadcasted_iota(jnp.int32, (1, _L), 1) + i * _L < N
    dis_ref[...] = jnp.where(mask, lax.rsqrt(deg), 0.0)
    u_ref[...] = jnp.where(mask, (deg - 1.0) / deg + cnt, 0.0)


def _tc_prep(dp, sp):
    f = pl.pallas_call(
        _prep_body,
        grid=(NP // _L,),
        in_specs=[
            pl.BlockSpec((NW, _L), lambda i: (0, i)),
            pl.BlockSpec((NW, _L), lambda i: (0, i)),
        ],
        out_specs=(pl.BlockSpec((1, _L), lambda i: (0, i)),
                   pl.BlockSpec((1, _L), lambda i: (0, i))),
        out_shape=(jax.ShapeDtypeStruct((1, NP), jnp.float32),
                   jax.ShapeDtypeStruct((1, NP), jnp.float32)),
    )
    return f(dp, sp)


def _scale_body(x, dis, xs_ref):
    xs_ref[...] = x[...] * dis[...]


def _tc_scale(x, dis):
    f = pl.pallas_call(
        _scale_body,
        grid=(NP // _L,),
        in_specs=[
            pl.BlockSpec((D, _L), lambda i: (0, i)),
            pl.BlockSpec((1, _L), lambda i: (0, i)),
        ],
        out_specs=pl.BlockSpec((D, _L), lambda i: (0, i)),
        out_shape=jax.ShapeDtypeStruct((D, NP), jnp.float32),
    )
    return f(x, dis)


def _mid_body(x, g, dis, u, phi, y_ref, ys_ref):
    sp = u[...] * x[...] - dis[...] * g[...]
    y = x[...] - phi[...] * sp
    y_ref[...] = y
    ys_ref[...] = dis[...] * y


def _tc_mid(x, g, dis, u, phi):
    f = pl.pallas_call(
        _mid_body,
        grid=(NP // _L,),
        in_specs=[
            pl.BlockSpec((D, _L), lambda i: (0, i)),
            pl.BlockSpec((D, _L), lambda i: (0, i)),
            pl.BlockSpec((1, _L), lambda i: (0, i)),
            pl.BlockSpec((1, _L), lambda i: (0, i)),
            pl.BlockSpec((D, 1), lambda i: (0, 0)),
        ],
        out_specs=(pl.BlockSpec((D, _L), lambda i: (0, i)),
                   pl.BlockSpec((D, _L), lambda i: (0, i))),
        out_shape=(jax.ShapeDtypeStruct((D, NP), jnp.float32),
                   jax.ShapeDtypeStruct((D, NP), jnp.float32)),
    )
    return f(x, g, dis, u, phi)


def _mm_body(relu, x, g, dis, u, phi, Wt, b, y_ref, ys_ref=None):
    z = x[...] - phi[...] * (u[...] * x[...] - dis[...] * g[...])
    y = jnp.dot(Wt[...], z, preferred_element_type=jnp.float32) + b[...]
    if relu:
        y = jnp.maximum(y, 0.0)
    y_ref[...] = y
    if ys_ref is not None:
        ys_ref[...] = dis[...] * y


def _tc_mm(x, g, dis, u, phi, Wt, b, relu, want_ys):
    blk = pl.BlockSpec((D, _L), lambda i: (0, i))
    out_specs = (blk, blk) if want_ys else blk
    out_shape = jax.ShapeDtypeStruct((D, NP), jnp.float32)
    if want_ys:
        out_shape = (out_shape, out_shape)
    f = pl.pallas_call(
        functools.partial(_mm_body, relu),
        grid=(NP // _L,),
        in_specs=[
            blk, blk,
            pl.BlockSpec((1, _L), lambda i: (0, i)),
            pl.BlockSpec((1, _L), lambda i: (0, i)),
            pl.BlockSpec((D, 1), lambda i: (0, 0)),
            pl.BlockSpec((D, D), lambda i: (0, 0)),
            pl.BlockSpec((D, 1), lambda i: (0, 0)),
        ],
        out_specs=out_specs,
        out_shape=out_shape,
    )
    return f(x, g, dis, u, phi, Wt, b)


@jax.jit
def _run(node_feat, edge_index, phi1, W1, b1, phi_hidden, phi2, W2, b2):
    xt = jnp.pad(node_feat, ((0, NP - N), (0, 0))).T
    rowp = jnp.pad(edge_index[0], (0, EP - E),
                   constant_values=NP - 1).reshape(NW * KCH, CH)
    colp = jnp.pad(edge_index[1], (0, EP - E),
                   constant_values=NP - 1).reshape(NW * KCH, CH)
    degp, selfp, packed = _sc_hist(rowp, colp)
    pk = packed.reshape(EP)
    dis, u = _tc_prep(degp, selfp)

    xs = _tc_scale(xt, dis)
    g = _sc_gs(xs.reshape(NW, FPT, NP), pk).reshape(D, NP)
    x1, xs = _tc_mm(xt, g, dis, u, phi1.reshape(D, 1), W1.T,
                    b1.reshape(D, 1), relu=True, want_ys=True)
    g = _sc_gs(xs.reshape(NW, FPT, NP), pk).reshape(D, NP)
    x2, xs = _tc_mid(x1, g, dis, u, phi_hidden[0].reshape(D, 1))
    g = _sc_gs(xs.reshape(NW, FPT, NP), pk).reshape(D, NP)
    x3, xs = _tc_mid(x2, g, dis, u, phi_hidden[1].reshape(D, 1))
    g = _sc_gs(xs.reshape(NW, FPT, NP), pk).reshape(D, NP)
    out = _tc_mm(x3, g, dis, u, phi2.reshape(D, 1), W2.T,
                 b2.reshape(D, 1), relu=False, want_ys=False)
    return out[:, :N].T


def kernel(node_feat, edge_index, phi1, W1, b1, phi_hidden, phi2, W2, b2):
    return _run(node_feat, edge_index, phi1, W1, b1, phi_hidden, phi2, W2, b2)


# speedup vs baseline: 1.2871x; 1.0020x over previous
import functools

import jax
import jax.numpy as jnp
from jax import lax
from jax.experimental import pallas as pl
from jax.experimental.pallas import tpu as pltpu
from jax.experimental.pallas import tpu_sc as plsc

N = 10000
NP = 10240
D = 128
E = 320000
CH = 128
NC = 2
NS = 16
NW = NC * NS
KCH = 80
EP = NW * KCH * CH
RPS = NP // NS
FPT = D // NW
GCH = 1024
NCHK = EP // GCH

_mesh = plsc.VectorSubcoreMesh(core_axis_name="c", subcore_axis_name="s")


@functools.partial(
    pl.kernel,
    out_type=(
        jax.ShapeDtypeStruct((NW, NP), jnp.float32),
        jax.ShapeDtypeStruct((NW, NP), jnp.float32),
        jax.ShapeDtypeStruct((NW * KCH, CH), jnp.int32),
    ),
    mesh=_mesh,
    scratch_types=[
        pltpu.VMEM((KCH, CH), jnp.int32),
        pltpu.VMEM((KCH, CH), jnp.int32),
        pltpu.VMEM((KCH, CH), jnp.int32),
        pltpu.VMEM((NP,), jnp.float32),
        pltpu.VMEM((NP,), jnp.float32),
    ],
    compiler_params=pltpu.CompilerParams(needs_layout_passes=False),
)
def _sc_hist(row_hbm, col_hbm, degp, selfp, packed,
             rows_v, cols_v, pk_v, deg_b, self_b):
    c = lax.axis_index("c")
    s = lax.axis_index("s")
    w = s * NC + c
    pltpu.sync_copy(row_hbm.at[pl.ds(w * KCH, KCH)], rows_v)
    pltpu.sync_copy(col_hbm.at[pl.ds(w * KCH, KCH)], cols_v)

    @pl.loop(0, NP // 16)
    def _(i):
        deg_b[pl.ds(i * 16, 16)] = jnp.zeros((16,), jnp.float32)
        self_b[pl.ds(i * 16, 16)] = jnp.zeros((16,), jnp.float32)

    ones = jnp.ones((16,), jnp.float32)

    @pl.loop(0, KCH)
    def _(k):
        for j in range(CH // 16):
            r = rows_v[k, pl.ds(j * 16, 16)]
            cc = cols_v[k, pl.ds(j * 16, 16)]
            plsc.addupdate_scatter(deg_b, [cc], ones)
            sel = jnp.where(r == cc, cc, NP - 1)
            plsc.addupdate_scatter(self_b, [sel], ones)
            pk_v[k, pl.ds(j * 16, 16)] = r * 16384 + cc

    pltpu.sync_copy(pk_v, packed.at[pl.ds(w * KCH, KCH)])
    pltpu.sync_copy(deg_b, degp.at[w])
    pltpu.sync_copy(self_b, selfp.at[w])


@functools.partial(
    pl.kernel,
    out_type=jax.ShapeDtypeStruct((NW, FPT, NP), jnp.float32),
    mesh=_mesh,
    scratch_types=[
        [pltpu.VMEM((NP,), jnp.float32)] * FPT,
        [pltpu.VMEM((NP,), jnp.float32)] * FPT,
        pltpu.VMEM((GCH,), jnp.int32),
        pltpu.VMEM((GCH,), jnp.int32),
        pltpu.SemaphoreType.DMA,
        pltpu.SemaphoreType.DMA,
        pltpu.SemaphoreType.DMA,
    ],
    compiler_params=pltpu.CompilerParams(needs_layout_passes=False),
)
def _sc_gs(xs_hbm, pk_hbm, gp, xs_v, acc_v, pb0, pb1, sx, s0, s1):
    c = lax.axis_index("c")
    s = lax.axis_index("s")
    w = s * NC + c
    dxs = [pltpu.async_copy(xs_hbm.at[w, f], xs_v[f], sx)
           for f in range(FPT)]
    d0 = pltpu.async_copy(pk_hbm.at[pl.ds(0, GCH)], pb0, s0)

    @pl.loop(0, NP // 16)
    def _(i):
        for f in range(FPT):
            acc_v[f][pl.ds(i * 16, 16)] = jnp.zeros((16,), jnp.float32)

    for d in dxs:
        d.wait()

    def process(buf):
        @pl.loop(0, GCH // 128, unroll=2)
        def _(q):
            ps = [buf[pl.ds(q * 128 + t * 16, 16)] for t in range(8)]
            ccs = [jnp.bitwise_and(p, 16383) for p in ps]
            rs = [lax.shift_right_logical(p, 14) for p in ps]
            for f in range(FPT):
                for t in range(8):
                    v = plsc.load_gather(xs_v[f], [ccs[t]])
                    plsc.addupdate_scatter(acc_v[f], [rs[t]], v)

    @pl.loop(0, NCHK // 2)
    def _(j):
        k = j * 2
        pltpu.make_async_copy(pk_hbm.at[pl.ds(0, GCH)], pb0, s0).wait()
        nxt = jnp.minimum((k + 1) * GCH, (NCHK - 1) * GCH)
        pltpu.async_copy(pk_hbm.at[pl.ds(nxt, GCH)], pb1, s1)
        process(pb0)
        pltpu.make_async_copy(pk_hbm.at[pl.ds(0, GCH)], pb1, s1).wait()
        nxt2 = jnp.minimum((k + 2) * GCH, (NCHK - 1) * GCH)
        pltpu.async_copy(pk_hbm.at[pl.ds(nxt2, GCH)], pb0, s0)
        process(pb1)

    pltpu.make_async_copy(pk_hbm.at[pl.ds(0, GCH)], pb0, s0).wait()
    for f in range(FPT):
        pltpu.sync_copy(acc_v[f], gp.at[w, f])


_L = 2048


def _prep_body(dp, sp, x, dis_ref, u_ref, xt_ref, xs_ref):
    i = pl.program_id(0)
    deg = jnp.sum(dp[...], axis=0, keepdims=True) + 1.0
    cnt = jnp.sum(sp[...], axis=0, keepdims=True)
    mask = lax.broadcasted_iota(jnp.int32, (1, _L), 1) + i * _L < N
    dis = jnp.where(mask, lax.rsqrt(deg), 0.0)
    dis_ref[...] = dis
    u_ref[...] = jnp.where(mask, (deg - 1.0) / deg + cnt, 0.0)
    xt = x[...].T
    xt_ref[...] = xt
    xs_ref[...] = xt * dis


def _tc_prep(dp, sp, x):
    f = pl.pallas_call(
        _prep_body,
        grid=(NP // _L,),
        in_specs=[
            pl.BlockSpec((NW, _L), lambda i: (0, i)),
            pl.BlockSpec((NW, _L), lambda i: (0, i)),
            pl.BlockSpec((_L, D), lambda i: (i, 0)),
        ],
        out_specs=(pl.BlockSpec((1, _L), lambda i: (0, i)),
                   pl.BlockSpec((1, _L), lambda i: (0, i)),
                   pl.BlockSpec((D, _L), lambda i: (0, i)),
                   pl.BlockSpec((D, _L), lambda i: (0, i))),
        out_shape=(jax.ShapeDtypeStruct((1, NP), jnp.float32),
                   jax.ShapeDtypeStruct((1, NP), jnp.float32),
                   jax.ShapeDtypeStruct((D, NP), jnp.float32),
                   jax.ShapeDtypeStruct((D, NP), jnp.float32)),
    )
    return f(dp, sp, x)


def _mid_body(x, g, dis, u, phi, y_ref, ys_ref):
    sp = u[...] * x[...] - dis[...] * g[...]
    y = x[...] - phi[...] * sp
    y_ref[...] = y
    ys_ref[...] = dis[...] * y


def _tc_mid(x, g, dis, u, phi):
    f = pl.pallas_call(
        _mid_body,
        grid=(NP // _L,),
        in_specs=[
            pl.BlockSpec((D, _L), lambda i: (0, i)),
            pl.BlockSpec((D, _L), lambda i: (0, i)),
            pl.BlockSpec((1, _L), lambda i: (0, i)),
            pl.BlockSpec((1, _L), lambda i: (0, i)),
            pl.BlockSpec((D, 1), lambda i: (0, 0)),
        ],
        out_specs=(pl.BlockSpec((D, _L), lambda i: (0, i)),
                   pl.BlockSpec((D, _L), lambda i: (0, i))),
        out_shape=(jax.ShapeDtypeStruct((D, NP), jnp.float32),
                   jax.ShapeDtypeStruct((D, NP), jnp.float32)),
    )
    return f(x, g, dis, u, phi)


def _mm_body(relu, x, g, dis, u, phi, Wt, b, y_ref, ys_ref=None):
    z = x[...] - phi[...] * (u[...] * x[...] - dis[...] * g[...])
    y = jnp.dot(Wt[...], z, preferred_element_type=jnp.float32) + b[...]
    if relu:
        y = jnp.maximum(y, 0.0)
        y_ref[...] = y
        ys_ref[...] = dis[...] * y
    else:
        y_ref[...] = y.T


def _tc_mm(x, g, dis, u, phi, Wt, b, relu, want_ys):
    blk = pl.BlockSpec((D, _L), lambda i: (0, i))
    if want_ys:
        out_specs = (blk, blk)
        out_shape = (jax.ShapeDtypeStruct((D, NP), jnp.float32),
                     jax.ShapeDtypeStruct((D, NP), jnp.float32))
    else:
        out_specs = pl.BlockSpec((_L, D), lambda i: (i, 0))
        out_shape = jax.ShapeDtypeStruct((NP, D), jnp.float32)
    f = pl.pallas_call(
        functools.partial(_mm_body, relu),
        grid=(NP // _L,),
        in_specs=[
            blk, blk,
            pl.BlockSpec((1, _L), lambda i: (0, i)),
            pl.BlockSpec((1, _L), lambda i: (0, i)),
            pl.BlockSpec((D, 1), lambda i: (0, 0)),
            pl.BlockSpec((D, D), lambda i: (0, 0)),
            pl.BlockSpec((D, 1), lambda i: (0, 0)),
        ],
        out_specs=out_specs,
        out_shape=out_shape,
    )
    return f(x, g, dis, u, phi, Wt, b)


@jax.jit
def _run(node_feat, edge_index, phi1, W1, b1, phi_hidden, phi2, W2, b2):
    xpad = jnp.pad(node_feat, ((0, NP - N), (0, 0)))
    rowp = jnp.pad(edge_index[0], (0, EP - E),
                   constant_values=NP - 1).reshape(NW * KCH, CH)
    colp = jnp.pad(edge_index[1], (0, EP - E),
                   constant_values=NP - 1).reshape(NW * KCH, CH)
    degp, selfp, packed = _sc_hist(rowp, colp)
    pk = packed.reshape(EP)
    dis, u, xt, xs = _tc_prep(degp, selfp, xpad)

    g = _sc_gs(xs.reshape(NW, FPT, NP), pk).reshape(D, NP)
    x1, xs = _tc_mm(xt, g, dis, u, phi1.reshape(D, 1), W1.T,
                    b1.reshape(D, 1), relu=True, want_ys=True)
    g = _sc_gs(xs.reshape(NW, FPT, NP), pk).reshape(D, NP)
    x2, xs = _tc_mid(x1, g, dis, u, phi_hidden[0].reshape(D, 1))
    g = _sc_gs(xs.reshape(NW, FPT, NP), pk).reshape(D, NP)
    x3, xs = _tc_mid(x2, g, dis, u, phi_hidden[1].reshape(D, 1))
    g = _sc_gs(xs.reshape(NW, FPT, NP), pk).reshape(D, NP)
    out = _tc_mm(x3, g, dis, u, phi2.reshape(D, 1), W2.T,
                 b2.reshape(D, 1), relu=False, want_ys=False)
    return out[:N]


def kernel(node_feat, edge_index, phi1, W1, b1, phi_hidden, phi2, W2, b2):
    return _run(node_feat, edge_index, phi1, W1, b1, phi_hidden, phi2, W2, b2)


# speedup vs baseline: 1.2893x; 1.0017x over previous
import functools

import jax
import jax.numpy as jnp
from jax import lax
from jax.experimental import pallas as pl
from jax.experimental.pallas import tpu as pltpu
from jax.experimental.pallas import tpu_sc as plsc

N = 10000
NP = 10240
D = 128
E = 320000
CH = 128
NC = 2
NS = 16
NW = NC * NS
KCH = 80
EP = NW * KCH * CH
RPS = NP // NS
FPT = D // NW
GCH = 2048
NCHK = EP // GCH

_mesh = plsc.VectorSubcoreMesh(core_axis_name="c", subcore_axis_name="s")


@functools.partial(
    pl.kernel,
    out_type=(
        jax.ShapeDtypeStruct((NW, NP), jnp.float32),
        jax.ShapeDtypeStruct((NW, NP), jnp.float32),
        jax.ShapeDtypeStruct((NW * KCH, CH), jnp.int32),
    ),
    mesh=_mesh,
    scratch_types=[
        pltpu.VMEM((KCH, CH), jnp.int32),
        pltpu.VMEM((KCH, CH), jnp.int32),
        pltpu.VMEM((KCH, CH), jnp.int32),
        pltpu.VMEM((NP,), jnp.float32),
        pltpu.VMEM((NP,), jnp.float32),
    ],
    compiler_params=pltpu.CompilerParams(needs_layout_passes=False),
)
def _sc_hist(row_hbm, col_hbm, degp, selfp, packed,
             rows_v, cols_v, pk_v, deg_b, self_b):
    c = lax.axis_index("c")
    s = lax.axis_index("s")
    w = s * NC + c
    pltpu.sync_copy(row_hbm.at[pl.ds(w * KCH, KCH)], rows_v)
    pltpu.sync_copy(col_hbm.at[pl.ds(w * KCH, KCH)], cols_v)

    @pl.loop(0, NP // 16)
    def _(i):
        deg_b[pl.ds(i * 16, 16)] = jnp.zeros((16,), jnp.float32)
        self_b[pl.ds(i * 16, 16)] = jnp.zeros((16,), jnp.float32)

    ones = jnp.ones((16,), jnp.float32)

    @pl.loop(0, KCH)
    def _(k):
        for j in range(CH // 16):
            r = rows_v[k, pl.ds(j * 16, 16)]
            cc = cols_v[k, pl.ds(j * 16, 16)]
            plsc.addupdate_scatter(deg_b, [cc], ones)
            sel = jnp.where(r == cc, cc, NP - 1)
            plsc.addupdate_scatter(self_b, [sel], ones)
            pk_v[k, pl.ds(j * 16, 16)] = r * 16384 + cc

    pltpu.sync_copy(pk_v, packed.at[pl.ds(w * KCH, KCH)])
    pltpu.sync_copy(deg_b, degp.at[w])
    pltpu.sync_copy(self_b, selfp.at[w])


@functools.partial(
    pl.kernel,
    out_type=jax.ShapeDtypeStruct((NW, FPT, NP), jnp.float32),
    mesh=_mesh,
    scratch_types=[
        [pltpu.VMEM((NP,), jnp.float32)] * FPT,
        [pltpu.VMEM((NP,), jnp.float32)] * FPT,
        pltpu.VMEM((GCH,), jnp.int32),
        pltpu.VMEM((GCH,), jnp.int32),
        pltpu.SemaphoreType.DMA,
        pltpu.SemaphoreType.DMA,
        pltpu.SemaphoreType.DMA,
    ],
    compiler_params=pltpu.CompilerParams(needs_layout_passes=False),
)
def _sc_gs(xs_hbm, pk_hbm, gp, xs_v, acc_v, pb0, pb1, sx, s0, s1):
    c = lax.axis_index("c")
    s = lax.axis_index("s")
    w = s * NC + c
    dxs = [pltpu.async_copy(xs_hbm.at[w, f], xs_v[f], sx)
           for f in range(FPT)]
    d0 = pltpu.async_copy(pk_hbm.at[pl.ds(0, GCH)], pb0, s0)

    @pl.loop(0, NP // 16)
    def _(i):
        for f in range(FPT):
            acc_v[f][pl.ds(i * 16, 16)] = jnp.zeros((16,), jnp.float32)

    for d in dxs:
        d.wait()

    def process(buf):
        @pl.loop(0, GCH // 128, unroll=2)
        def _(q):
            ps = [buf[pl.ds(q * 128 + t * 16, 16)] for t in range(8)]
            ccs = [jnp.bitwise_and(p, 16383) for p in ps]
            rs = [lax.shift_right_logical(p, 14) for p in ps]
            for f in range(FPT):
                for t in range(8):
                    v = plsc.load_gather(xs_v[f], [ccs[t]])
                    plsc.addupdate_scatter(acc_v[f], [rs[t]], v)

    @pl.loop(0, NCHK // 2)
    def _(j):
        k = j * 2
        pltpu.make_async_copy(pk_hbm.at[pl.ds(0, GCH)], pb0, s0).wait()
        nxt = jnp.minimum((k + 1) * GCH, (NCHK - 1) * GCH)
        pltpu.async_copy(pk_hbm.at[pl.ds(nxt, GCH)], pb1, s1)
        process(pb0)
        pltpu.make_async_copy(pk_hbm.at[pl.ds(0, GCH)], pb1, s1).wait()
        nxt2 = jnp.minimum((k + 2) * GCH, (NCHK - 1) * GCH)
        pltpu.async_copy(pk_hbm.at[pl.ds(nxt2, GCH)], pb0, s0)
        process(pb1)

    pltpu.make_async_copy(pk_hbm.at[pl.ds(0, GCH)], pb0, s0).wait()
    for f in range(FPT):
        pltpu.sync_copy(acc_v[f], gp.at[w, f])


_L = 2048


def _prep_body(dp, sp, x, dis_ref, u_ref, xt_ref, xs_ref):
    i = pl.program_id(0)
    deg = jnp.sum(dp[...], axis=0, keepdims=True) + 1.0
    cnt = jnp.sum(sp[...], axis=0, keepdims=True)
    mask = lax.broadcasted_iota(jnp.int32, (1, _L), 1) + i * _L < N
    dis = jnp.where(mask, lax.rsqrt(deg), 0.0)
    dis_ref[...] = dis
    u_ref[...] = jnp.where(mask, (deg - 1.0) / deg + cnt, 0.0)
    xt = x[...].T
    xt_ref[...] = xt
    xs_ref[...] = xt * dis


def _tc_prep(dp, sp, x):
    f = pl.pallas_call(
        _prep_body,
        grid=(NP // _L,),
        in_specs=[
            pl.BlockSpec((NW, _L), lambda i: (0, i)),
            pl.BlockSpec((NW, _L), lambda i: (0, i)),
            pl.BlockSpec((_L, D), lambda i: (i, 0)),
        ],
        out_specs=(pl.BlockSpec((1, _L), lambda i: (0, i)),
                   pl.BlockSpec((1, _L), lambda i: (0, i)),
                   pl.BlockSpec((D, _L), lambda i: (0, i)),
                   pl.BlockSpec((D, _L), lambda i: (0, i))),
        out_shape=(jax.ShapeDtypeStruct((1, NP), jnp.float32),
                   jax.ShapeDtypeStruct((1, NP), jnp.float32),
                   jax.ShapeDtypeStruct((D, NP), jnp.float32),
                   jax.ShapeDtypeStruct((D, NP), jnp.float32)),
    )
    return f(dp, sp, x)


def _mid_body(x, g, dis, u, phi, y_ref, ys_ref):
    sp = u[...] * x[...] - dis[...] * g[...]
    y = x[...] - phi[...] * sp
    y_ref[...] = y
    ys_ref[...] = dis[...] * y


def _tc_mid(x, g, dis, u, phi):
    f = pl.pallas_call(
        _mid_body,
        grid=(NP // _L,),
        in_specs=[
            pl.BlockSpec((D, _L), lambda i: (0, i)),
            pl.BlockSpec((D, _L), lambda i: (0, i)),
            pl.BlockSpec((1, _L), lambda i: (0, i)),
            pl.BlockSpec((1, _L), lambda i: (0, i)),
            pl.BlockSpec((D, 1), lambda i: (0, 0)),
        ],
        out_specs=(pl.BlockSpec((D, _L), lambda i: (0, i)),
                   pl.BlockSpec((D, _L), lambda i: (0, i))),
        out_shape=(jax.ShapeDtypeStruct((D, NP), jnp.float32),
                   jax.ShapeDtypeStruct((D, NP), jnp.float32)),
    )
    return f(x, g, dis, u, phi)


def _mm_body(relu, x, g, dis, u, phi, Wt, b, y_ref, ys_ref=None):
    z = x[...] - phi[...] * (u[...] * x[...] - dis[...] * g[...])
    y = jnp.dot(Wt[...], z, preferred_element_type=jnp.float32) + b[...]
    if relu:
        y = jnp.maximum(y, 0.0)
        y_ref[...] = y
        ys_ref[...] = dis[...] * y
    else:
        y_ref[...] = y.T


def _tc_mm(x, g, dis, u, phi, Wt, b, relu, want_ys):
    blk = pl.BlockSpec((D, _L), lambda i: (0, i))
    if want_ys:
        out_specs = (blk, blk)
        out_shape = (jax.ShapeDtypeStruct((D, NP), jnp.float32),
                     jax.ShapeDtypeStruct((D, NP), jnp.float32))
    else:
        out_specs = pl.BlockSpec((_L, D), lambda i: (i, 0))
        out_shape = jax.ShapeDtypeStruct((NP, D), jnp.float32)
    f = pl.pallas_call(
        functools.partial(_mm_body, relu),
        grid=(NP // _L,),
        in_specs=[
            blk, blk,
            pl.BlockSpec((1, _L), lambda i: (0, i)),
            pl.BlockSpec((1, _L), lambda i: (0, i)),
            pl.BlockSpec((D, 1), lambda i: (0, 0)),
            pl.BlockSpec((D, D), lambda i: (0, 0)),
            pl.BlockSpec((D, 1), lambda i: (0, 0)),
        ],
        out_specs=out_specs,
        out_shape=out_shape,
    )
    return f(x, g, dis, u, phi, Wt, b)


@jax.jit
def _run(node_feat, edge_index, phi1, W1, b1, phi_hidden, phi2, W2, b2):
    xpad = jnp.pad(node_feat, ((0, NP - N), (0, 0)))
    rowp = jnp.pad(edge_index[0], (0, EP - E),
                   constant_values=NP - 1).reshape(NW * KCH, CH)
    colp = jnp.pad(edge_index[1], (0, EP - E),
                   constant_values=NP - 1).reshape(NW * KCH, CH)
    degp, selfp, packed = _sc_hist(rowp, colp)
    pk = packed.reshape(EP)
    dis, u, xt, xs = _tc_prep(degp, selfp, xpad)

    g = _sc_gs(xs.reshape(NW, FPT, NP), pk).reshape(D, NP)
    x1, xs = _tc_mm(xt, g, dis, u, phi1.reshape(D, 1), W1.T,
                    b1.reshape(D, 1), relu=True, want_ys=True)
    g = _sc_gs(xs.reshape(NW, FPT, NP), pk).reshape(D, NP)
    x2, xs = _tc_mid(x1, g, dis, u, phi_hidden[0].reshape(D, 1))
    g = _sc_gs(xs.reshape(NW, FPT, NP), pk).reshape(D, NP)
    x3, xs = _tc_mid(x2, g, dis, u, phi_hidden[1].reshape(D, 1))
    g = _sc_gs(xs.reshape(NW, FPT, NP), pk).reshape(D, NP)
    out = _tc_mm(x3, g, dis, u, phi2.reshape(D, 1), W2.T,
                 b2.reshape(D, 1), relu=False, want_ys=False)
    return out[:N]


def kernel(node_feat, edge_index, phi1, W1, b1, phi_hidden, phi2, W2, b2):
    return _run(node_feat, edge_index, phi1, W1, b1, phi_hidden, phi2, W2, b2)


# speedup vs baseline: 1.3027x; 1.0104x over previous
import functools

import jax
import jax.numpy as jnp
from jax import lax
from jax.experimental import pallas as pl
from jax.experimental.pallas import tpu as pltpu
from jax.experimental.pallas import tpu_sc as plsc

N = 10000
NP = 10240
D = 128
E = 320000
CH = 128
NC = 2
NS = 16
NW = NC * NS
KCH = 80
EP = NW * KCH * CH
RPS = NP // NS
FPT = D // NW
GCH = 2048
NCHK = EP // GCH

_mesh = plsc.VectorSubcoreMesh(core_axis_name="c", subcore_axis_name="s")


@functools.partial(
    pl.kernel,
    out_type=(
        jax.ShapeDtypeStruct((NW, NP), jnp.float32),
        jax.ShapeDtypeStruct((NW, NP), jnp.float32),
        jax.ShapeDtypeStruct((NW * KCH, CH), jnp.int32),
    ),
    mesh=_mesh,
    scratch_types=[
        pltpu.VMEM((KCH, CH), jnp.int32),
        pltpu.VMEM((KCH, CH), jnp.int32),
        pltpu.VMEM((KCH, CH), jnp.int32),
        pltpu.VMEM((NP,), jnp.float32),
        pltpu.VMEM((NP,), jnp.float32),
    ],
    compiler_params=pltpu.CompilerParams(needs_layout_passes=False),
)
def _sc_hist(row_hbm, col_hbm, degp, selfp, packed,
             rows_v, cols_v, pk_v, deg_b, self_b):
    c = lax.axis_index("c")
    s = lax.axis_index("s")
    w = s * NC + c
    pltpu.sync_copy(row_hbm.at[pl.ds(w * KCH, KCH)], rows_v)
    pltpu.sync_copy(col_hbm.at[pl.ds(w * KCH, KCH)], cols_v)

    @pl.loop(0, NP // 16)
    def _(i):
        deg_b[pl.ds(i * 16, 16)] = jnp.zeros((16,), jnp.float32)
        self_b[pl.ds(i * 16, 16)] = jnp.zeros((16,), jnp.float32)

    ones = jnp.ones((16,), jnp.float32)

    @pl.loop(0, KCH)
    def _(k):
        for j in range(CH // 16):
            r = rows_v[k, pl.ds(j * 16, 16)]
            cc = cols_v[k, pl.ds(j * 16, 16)]
            plsc.addupdate_scatter(deg_b, [cc], ones)
            sel = jnp.where(r == cc, cc, NP - 1)
            plsc.addupdate_scatter(self_b, [sel], ones)
            pk_v[k, pl.ds(j * 16, 16)] = r * 16384 + cc

    pltpu.sync_copy(pk_v, packed.at[pl.ds(w * KCH, KCH)])
    pltpu.sync_copy(deg_b, degp.at[w])
    pltpu.sync_copy(self_b, selfp.at[w])


@functools.partial(
    pl.kernel,
    out_type=jax.ShapeDtypeStruct((NW, FPT, NP), jnp.float32),
    mesh=_mesh,
    scratch_types=[
        [pltpu.VMEM((NP,), jnp.float32)] * FPT,
        [pltpu.VMEM((NP,), jnp.float32)] * FPT,
        pltpu.VMEM((GCH,), jnp.int32),
        pltpu.VMEM((GCH,), jnp.int32),
        pltpu.SemaphoreType.DMA,
        pltpu.SemaphoreType.DMA,
        pltpu.SemaphoreType.DMA,
    ],
    compiler_params=pltpu.CompilerParams(needs_layout_passes=False),
)
def _sc_gs(xs_hbm, pk_hbm, gp, xs_v, acc_v, pb0, pb1, sx, s0, s1):
    c = lax.axis_index("c")
    s = lax.axis_index("s")
    w = s * NC + c
    dxs = [pltpu.async_copy(xs_hbm.at[w, f], xs_v[f], sx)
           for f in range(FPT)]
    d0 = pltpu.async_copy(pk_hbm.at[pl.ds(0, GCH)], pb0, s0)

    @pl.loop(0, NP // 16)
    def _(i):
        for f in range(FPT):
            acc_v[f][pl.ds(i * 16, 16)] = jnp.zeros((16,), jnp.float32)

    for d in dxs:
        d.wait()

    def process(buf):
        @pl.loop(0, GCH // 256, unroll=1)
        def _(q):
            ps = [buf[pl.ds(q * 256 + t * 16, 16)] for t in range(16)]
            ccs = [jnp.bitwise_and(p, 16383) for p in ps]
            rs = [lax.shift_right_logical(p, 14) for p in ps]
            for f in range(FPT):
                for t in range(16):
                    v = plsc.load_gather(xs_v[f], [ccs[t]])
                    plsc.addupdate_scatter(acc_v[f], [rs[t]], v)

    @pl.loop(0, NCHK // 2)
    def _(j):
        k = j * 2
        pltpu.make_async_copy(pk_hbm.at[pl.ds(0, GCH)], pb0, s0).wait()
        nxt = jnp.minimum((k + 1) * GCH, (NCHK - 1) * GCH)
        pltpu.async_copy(pk_hbm.at[pl.ds(nxt, GCH)], pb1, s1)
        process(pb0)
        pltpu.make_async_copy(pk_hbm.at[pl.ds(0, GCH)], pb1, s1).wait()
        nxt2 = jnp.minimum((k + 2) * GCH, (NCHK - 1) * GCH)
        pltpu.async_copy(pk_hbm.at[pl.ds(nxt2, GCH)], pb0, s0)
        process(pb1)

    pltpu.make_async_copy(pk_hbm.at[pl.ds(0, GCH)], pb0, s0).wait()
    for f in range(FPT):
        pltpu.sync_copy(acc_v[f], gp.at[w, f])


_L = 2048


def _prep_body(dp, sp, x, dis_ref, u_ref, xt_ref, xs_ref):
    i = pl.program_id(0)
    deg = jnp.sum(dp[...], axis=0, keepdims=True) + 1.0
    cnt = jnp.sum(sp[...], axis=0, keepdims=True)
    mask = lax.broadcasted_iota(jnp.int32, (1, _L), 1) + i * _L < N
    dis = jnp.where(mask, lax.rsqrt(deg), 0.0)
    dis_ref[...] = dis
    u_ref[...] = jnp.where(mask, (deg - 1.0) / deg + cnt, 0.0)
    xt = x[...].T
    xt_ref[...] = xt
    xs_ref[...] = xt * dis


def _tc_prep(dp, sp, x):
    f = pl.pallas_call(
        _prep_body,
        grid=(NP // _L,),
        in_specs=[
            pl.BlockSpec((NW, _L), lambda i: (0, i)),
            pl.BlockSpec((NW, _L), lambda i: (0, i)),
            pl.BlockSpec((_L, D), lambda i: (i, 0)),
        ],
        out_specs=(pl.BlockSpec((1, _L), lambda i: (0, i)),
                   pl.BlockSpec((1, _L), lambda i: (0, i)),
                   pl.BlockSpec((D, _L), lambda i: (0, i)),
                   pl.BlockSpec((D, _L), lambda i: (0, i))),
        out_shape=(jax.ShapeDtypeStruct((1, NP), jnp.float32),
                   jax.ShapeDtypeStruct((1, NP), jnp.float32),
                   jax.ShapeDtypeStruct((D, NP), jnp.float32),
                   jax.ShapeDtypeStruct((D, NP), jnp.float32)),
    )
    return f(dp, sp, x)


def _mid_body(x, g, dis, u, phi, y_ref, ys_ref):
    sp = u[...] * x[...] - dis[...] * g[...]
    y = x[...] - phi[...] * sp
    y_ref[...] = y
    ys_ref[...] = dis[...] * y


def _tc_mid(x, g, dis, u, phi):
    f = pl.pallas_call(
        _mid_body,
        grid=(NP // _L,),
        in_specs=[
            pl.BlockSpec((D, _L), lambda i: (0, i)),
            pl.BlockSpec((D, _L), lambda i: (0, i)),
            pl.BlockSpec((1, _L), lambda i: (0, i)),
            pl.BlockSpec((1, _L), lambda i: (0, i)),
            pl.BlockSpec((D, 1), lambda i: (0, 0)),
        ],
        out_specs=(pl.BlockSpec((D, _L), lambda i: (0, i)),
                   pl.BlockSpec((D, _L), lambda i: (0, i))),
        out_shape=(jax.ShapeDtypeStruct((D, NP), jnp.float32),
                   jax.ShapeDtypeStruct((D, NP), jnp.float32)),
    )
    return f(x, g, dis, u, phi)


def _mm_body(relu, x, g, dis, u, phi, Wt, b, y_ref, ys_ref=None):
    z = x[...] - phi[...] * (u[...] * x[...] - dis[...] * g[...])
    y = jnp.dot(Wt[...], z, preferred_element_type=jnp.float32) + b[...]
    if relu:
        y = jnp.maximum(y, 0.0)
        y_ref[...] = y
        ys_ref[...] = dis[...] * y
    else:
        y_ref[...] = y.T


def _tc_mm(x, g, dis, u, phi, Wt, b, relu, want_ys):
    blk = pl.BlockSpec((D, _L), lambda i: (0, i))
    if want_ys:
        out_specs = (blk, blk)
        out_shape = (jax.ShapeDtypeStruct((D, NP), jnp.float32),
                     jax.ShapeDtypeStruct((D, NP), jnp.float32))
    else:
        out_specs = pl.BlockSpec((_L, D), lambda i: (i, 0))
        out_shape = jax.ShapeDtypeStruct((NP, D), jnp.float32)
    f = pl.pallas_call(
        functools.partial(_mm_body, relu),
        grid=(NP // _L,),
        in_specs=[
            blk, blk,
            pl.BlockSpec((1, _L), lambda i: (0, i)),
            pl.BlockSpec((1, _L), lambda i: (0, i)),
            pl.BlockSpec((D, 1), lambda i: (0, 0)),
            pl.BlockSpec((D, D), lambda i: (0, 0)),
            pl.BlockSpec((D, 1), lambda i: (0, 0)),
        ],
        out_specs=out_specs,
        out_shape=out_shape,
    )
    return f(x, g, dis, u, phi, Wt, b)


@jax.jit
def _run(node_feat, edge_index, phi1, W1, b1, phi_hidden, phi2, W2, b2):
    xpad = jnp.pad(node_feat, ((0, NP - N), (0, 0)))
    rowp = jnp.pad(edge_index[0], (0, EP - E),
                   constant_values=NP - 1).reshape(NW * KCH, CH)
    colp = jnp.pad(edge_index[1], (0, EP - E),
                   constant_values=NP - 1).reshape(NW * KCH, CH)
    degp, selfp, packed = _sc_hist(rowp, colp)
    pk = packed.reshape(EP)
    dis, u, xt, xs = _tc_prep(degp, selfp, xpad)

    g = _sc_gs(xs.reshape(NW, FPT, NP), pk).reshape(D, NP)
    x1, xs = _tc_mm(xt, g, dis, u, phi1.reshape(D, 1), W1.T,
                    b1.reshape(D, 1), relu=True, want_ys=True)
    g = _sc_gs(xs.reshape(NW, FPT, NP), pk).reshape(D, NP)
    x2, xs = _tc_mid(x1, g, dis, u, phi_hidden[0].reshape(D, 1))
    g = _sc_gs(xs.reshape(NW, FPT, NP), pk).reshape(D, NP)
    x3, xs = _tc_mid(x2, g, dis, u, phi_hidden[1].reshape(D, 1))
    g = _sc_gs(xs.reshape(NW, FPT, NP), pk).reshape(D, NP)
    out = _tc_mm(x3, g, dis, u, phi2.reshape(D, 1), W2.T,
                 b2.reshape(D, 1), relu=False, want_ys=False)
    return out[:N]


def kernel(node_feat, edge_index, phi1, W1, b1, phi_hidden, phi2, W2, b2):
    return _run(node_feat, edge_index, phi1, W1, b1, phi_hidden, phi2, W2, b2)


# speedup vs baseline: 1.3037x; 1.0007x over previous
import functools

import jax
import jax.numpy as jnp
from jax import lax
from jax.experimental import pallas as pl
from jax.experimental.pallas import tpu as pltpu
from jax.experimental.pallas import tpu_sc as plsc

N = 10000
NP = 10240
D = 128
E = 320000
CH = 128
NC = 2
NS = 16
NW = NC * NS
KCH = 80
EP = NW * KCH * CH
RPS = NP // NS
FPT = D // NW
GCH = 2048
NCHK = EP // GCH

_mesh = plsc.VectorSubcoreMesh(core_axis_name="c", subcore_axis_name="s")


@functools.partial(
    pl.kernel,
    out_type=(
        jax.ShapeDtypeStruct((NW, NP), jnp.float32),
        jax.ShapeDtypeStruct((NW, NP), jnp.float32),
        jax.ShapeDtypeStruct((NW * KCH, CH), jnp.int32),
    ),
    mesh=_mesh,
    scratch_types=[
        pltpu.VMEM((KCH, CH), jnp.int32),
        pltpu.VMEM((KCH, CH), jnp.int32),
        pltpu.VMEM((KCH, CH), jnp.int32),
        pltpu.VMEM((NP,), jnp.float32),
        pltpu.VMEM((NP,), jnp.float32),
    ],
    compiler_params=pltpu.CompilerParams(needs_layout_passes=False),
)
def _sc_hist(row_hbm, col_hbm, degp, selfp, packed,
             rows_v, cols_v, pk_v, deg_b, self_b):
    c = lax.axis_index("c")
    s = lax.axis_index("s")
    w = s * NC + c
    pltpu.sync_copy(row_hbm.at[pl.ds(w * KCH, KCH)], rows_v)
    pltpu.sync_copy(col_hbm.at[pl.ds(w * KCH, KCH)], cols_v)

    @pl.loop(0, NP // 16)
    def _(i):
        deg_b[pl.ds(i * 16, 16)] = jnp.zeros((16,), jnp.float32)
        self_b[pl.ds(i * 16, 16)] = jnp.zeros((16,), jnp.float32)

    ones = jnp.ones((16,), jnp.float32)

    @pl.loop(0, KCH)
    def _(k):
        for j in range(CH // 16):
            r = rows_v[k, pl.ds(j * 16, 16)]
            cc = cols_v[k, pl.ds(j * 16, 16)]
            plsc.addupdate_scatter(deg_b, [cc], ones)
            sel = jnp.where(r == cc, cc, NP - 1)
            plsc.addupdate_scatter(self_b, [sel], ones)
            pk_v[k, pl.ds(j * 16, 16)] = r * 16384 + cc

    pltpu.sync_copy(pk_v, packed.at[pl.ds(w * KCH, KCH)])
    pltpu.sync_copy(deg_b, degp.at[w])
    pltpu.sync_copy(self_b, selfp.at[w])


@functools.partial(
    pl.kernel,
    out_type=jax.ShapeDtypeStruct((NW, FPT, NP), jnp.float32),
    mesh=_mesh,
    scratch_types=[
        [pltpu.VMEM((NP,), jnp.float32)] * FPT,
        [pltpu.VMEM((NP,), jnp.float32)] * FPT,
        pltpu.VMEM((GCH,), jnp.int32),
        pltpu.VMEM((GCH,), jnp.int32),
        pltpu.SemaphoreType.DMA,
        pltpu.SemaphoreType.DMA,
        pltpu.SemaphoreType.DMA,
    ],
    compiler_params=pltpu.CompilerParams(needs_layout_passes=False),
)
def _sc_gs(xs_hbm, pk_hbm, gp, xs_v, acc_v, pb0, pb1, sx, s0, s1):
    c = lax.axis_index("c")
    s = lax.axis_index("s")
    w = s * NC + c
    dxs = [pltpu.async_copy(xs_hbm.at[w, f], xs_v[f], sx)
           for f in range(FPT)]
    d0 = pltpu.async_copy(pk_hbm.at[pl.ds(0, GCH)], pb0, s0)

    @pl.loop(0, NP // 16)
    def _(i):
        for f in range(FPT):
            acc_v[f][pl.ds(i * 16, 16)] = jnp.zeros((16,), jnp.float32)

    for d in dxs:
        d.wait()

    def process(buf):
        @pl.loop(0, GCH // 256, unroll=2)
        def _(q):
            ps = [buf[pl.ds(q * 256 + t * 16, 16)] for t in range(16)]
            ccs = [jnp.bitwise_and(p, 16383) for p in ps]
            rs = [lax.shift_right_logical(p, 14) for p in ps]
            for f in range(FPT):
                for t in range(16):
                    v = plsc.load_gather(xs_v[f], [ccs[t]])
                    plsc.addupdate_scatter(acc_v[f], [rs[t]], v)

    @pl.loop(0, NCHK // 2)
    def _(j):
        k = j * 2
        pltpu.make_async_copy(pk_hbm.at[pl.ds(0, GCH)], pb0, s0).wait()
        nxt = jnp.minimum((k + 1) * GCH, (NCHK - 1) * GCH)
        pltpu.async_copy(pk_hbm.at[pl.ds(nxt, GCH)], pb1, s1)
        process(pb0)
        pltpu.make_async_copy(pk_hbm.at[pl.ds(0, GCH)], pb1, s1).wait()
        nxt2 = jnp.minimum((k + 2) * GCH, (NCHK - 1) * GCH)
        pltpu.async_copy(pk_hbm.at[pl.ds(nxt2, GCH)], pb0, s0)
        process(pb1)

    pltpu.make_async_copy(pk_hbm.at[pl.ds(0, GCH)], pb0, s0).wait()
    for f in range(FPT):
        pltpu.sync_copy(acc_v[f], gp.at[w, f])


_L = 2048


def _prep_body(dp, sp, x, dis_ref, u_ref, xt_ref, xs_ref):
    i = pl.program_id(0)
    deg = jnp.sum(dp[...], axis=0, keepdims=True) + 1.0
    cnt = jnp.sum(sp[...], axis=0, keepdims=True)
    mask = lax.broadcasted_iota(jnp.int32, (1, _L), 1) + i * _L < N
    dis = jnp.where(mask, lax.rsqrt(deg), 0.0)
    dis_ref[...] = dis
    u_ref[...] = jnp.where(mask, (deg - 1.0) / deg + cnt, 0.0)
    xt = x[...].T
    xt_ref[...] = xt
    xs_ref[...] = xt * dis


def _tc_prep(dp, sp, x):
    f = pl.pallas_call(
        _prep_body,
        grid=(NP // _L,),
        in_specs=[
            pl.BlockSpec((NW, _L), lambda i: (0, i)),
            pl.BlockSpec((NW, _L), lambda i: (0, i)),
            pl.BlockSpec((_L, D), lambda i: (i, 0)),
        ],
        out_specs=(pl.BlockSpec((1, _L), lambda i: (0, i)),
                   pl.BlockSpec((1, _L), lambda i: (0, i)),
                   pl.BlockSpec((D, _L), lambda i: (0, i)),
                   pl.BlockSpec((D, _L), lambda i: (0, i))),
        out_shape=(jax.ShapeDtypeStruct((1, NP), jnp.float32),
                   jax.ShapeDtypeStruct((1, NP), jnp.float32),
                   jax.ShapeDtypeStruct((D, NP), jnp.float32),
                   jax.ShapeDtypeStruct((D, NP), jnp.float32)),
    )
    return f(dp, sp, x)


def _mid_body(x, g, dis, u, phi, y_ref, ys_ref):
    sp = u[...] * x[...] - dis[...] * g[...]
    y = x[...] - phi[...] * sp
    y_ref[...] = y
    ys_ref[...] = dis[...] * y


def _tc_mid(x, g, dis, u, phi):
    f = pl.pallas_call(
        _mid_body,
        grid=(NP // _L,),
        in_specs=[
            pl.BlockSpec((D, _L), lambda i: (0, i)),
            pl.BlockSpec((D, _L), lambda i: (0, i)),
            pl.BlockSpec((1, _L), lambda i: (0, i)),
            pl.BlockSpec((1, _L), lambda i: (0, i)),
            pl.BlockSpec((D, 1), lambda i: (0, 0)),
        ],
        out_specs=(pl.BlockSpec((D, _L), lambda i: (0, i)),
                   pl.BlockSpec((D, _L), lambda i: (0, i))),
        out_shape=(jax.ShapeDtypeStruct((D, NP), jnp.float32),
                   jax.ShapeDtypeStruct((D, NP), jnp.float32)),
    )
    return f(x, g, dis, u, phi)


def _mm_body(relu, x, g, dis, u, phi, Wt, b, y_ref, ys_ref=None):
    z = x[...] - phi[...] * (u[...] * x[...] - dis[...] * g[...])
    y = jnp.dot(Wt[...], z, preferred_element_type=jnp.float32) + b[...]
    if relu:
        y = jnp.maximum(y, 0.0)
        y_ref[...] = y
        ys_ref[...] = dis[...] * y
    else:
        y_ref[...] = y.T


def _tc_mm(x, g, dis, u, phi, Wt, b, relu, want_ys):
    blk = pl.BlockSpec((D, _L), lambda i: (0, i))
    if want_ys:
        out_specs = (blk, blk)
        out_shape = (jax.ShapeDtypeStruct((D, NP), jnp.float32),
                     jax.ShapeDtypeStruct((D, NP), jnp.float32))
    else:
        out_specs = pl.BlockSpec((_L, D), lambda i: (i, 0))
        out_shape = jax.ShapeDtypeStruct((NP, D), jnp.float32)
    f = pl.pallas_call(
        functools.partial(_mm_body, relu),
        grid=(NP // _L,),
        in_specs=[
            blk, blk,
            pl.BlockSpec((1, _L), lambda i: (0, i)),
            pl.BlockSpec((1, _L), lambda i: (0, i)),
            pl.BlockSpec((D, 1), lambda i: (0, 0)),
            pl.BlockSpec((D, D), lambda i: (0, 0)),
            pl.BlockSpec((D, 1), lambda i: (0, 0)),
        ],
        out_specs=out_specs,
        out_shape=out_shape,
    )
    return f(x, g, dis, u, phi, Wt, b)


@jax.jit
def _run(node_feat, edge_index, phi1, W1, b1, phi_hidden, phi2, W2, b2):
    xpad = jnp.pad(node_feat, ((0, NP - N), (0, 0)))
    rowp = jnp.pad(edge_index[0], (0, EP - E),
                   constant_values=NP - 1).reshape(NW * KCH, CH)
    colp = jnp.pad(edge_index[1], (0, EP - E),
                   constant_values=NP - 1).reshape(NW * KCH, CH)
    degp, selfp, packed = _sc_hist(rowp, colp)
    pk = packed.reshape(EP)
    dis, u, xt, xs = _tc_prep(degp, selfp, xpad)

    g = _sc_gs(xs.reshape(NW, FPT, NP), pk).reshape(D, NP)
    x1, xs = _tc_mm(xt, g, dis, u, phi1.reshape(D, 1), W1.T,
                    b1.reshape(D, 1), relu=True, want_ys=True)
    g = _sc_gs(xs.reshape(NW, FPT, NP), pk).reshape(D, NP)
    x2, xs = _tc_mid(x1, g, dis, u, phi_hidden[0].reshape(D, 1))
    g = _sc_gs(xs.reshape(NW, FPT, NP), pk).reshape(D, NP)
    x3, xs = _tc_mid(x2, g, dis, u, phi_hidden[1].reshape(D, 1))
    g = _sc_gs(xs.reshape(NW, FPT, NP), pk).reshape(D, NP)
    out = _tc_mm(x3, g, dis, u, phi2.reshape(D, 1), W2.T,
                 b2.reshape(D, 1), relu=False, want_ys=False)
    return out[:N]


def kernel(node_feat, edge_index, phi1, W1, b1, phi_hidden, phi2, W2, b2):
    return _run(node_feat, edge_index, phi1, W1, b1, phi_hidden, phi2, W2, b2)
